# Initial kernel scaffold; baseline (speedup 1.0000x reference)
import jax
import jax.numpy as jnp
from jax.experimental import pallas as pl


def kernel(word_ids, topic_ids, ww_src, ww_dst, ww_w, wt_src, wt_dst, wt_w, wd_src, wd_dst, wd_w, td_src, td_dst, td_w, tt_src, tt_dst, tt_w, doc_graph_ids, y_data, word_embeds, topic_embeds, adapt_W, adapt_b, conv_W, conv_b, out_W, out_b):
    raise NotImplementedError("write your pallas kernel here")


# speedup vs baseline: 1.7649x; 1.7649x over previous
import functools

import jax
import jax.numpy as jnp
from jax import lax
from jax.experimental import pallas as pl
from jax.experimental.pallas import tpu as pltpu
from jax.experimental.pallas import tpu_sc as plsc

HD = 128
NWN = 50000
NTN = 3200
NDN = 6400
NB = 64
VOC = 100000
HI = 300

WW_E, WT_E, WD_E, TD_E, TT_E = 323584, 81920, 163840, 40960, 20480
GROWS = 53248
CW = 50176
WRANGE = 12544
CT = 3328
CD = 6656
WIN = 128

_mesh = plsc.VectorSubcoreMesh(core_axis_name="c", subcore_axis_name="s")


def _fill_zero_2d(ref, nrows):
    def body(i, _):
        for c in range(HD // 16):
            ref[i, pl.ds(16 * c, 16)] = jnp.zeros((16,), jnp.float32)
        return 0
    lax.fori_loop(0, nrows, body, 0)


def _fill_zero_1d(ref, n):
    def body(i, _):
        ref[pl.ds(16 * i, 16)] = jnp.zeros((16,), jnp.float32)
        return 0
    lax.fori_loop(0, n // 16, body, 0)


def _fill_ones_1d(ref, n):
    for i in range(n // 16):
        ref[pl.ds(16 * i, 16)] = jnp.ones((16,), jnp.float32)


def _zero_shared_rows(acc_sh, zbuf, row0, nrows):
    off = 0
    while off < nrows:
        n = min(128, nrows - off)
        pltpu.sync_copy(zbuf.at[pl.ds(0, n)], acc_sh.at[pl.ds(row0 + off, n)])
        off += n


def _count_pass(dst_hbm, cnt_sh, ones, dstb, n_edges, sid):
    share = n_edges // 16
    nwin = share // WIN

    def body(w, _):
        base = sid * share + w * WIN
        pltpu.sync_copy(dst_hbm.at[pl.ds(base, WIN)], dstb)
        pltpu.sync_copy(ones, cnt_sh.at[dstb], add=True)
        return 0
    lax.fori_loop(0, nwin, body, 0)


def _edge_pass(src_hbm, dst_hbm, w_hbm, h_hbm, acc_sh,
               srcb, dstb, wb, gb, rows, sem, n_edges, wid, nsplit,
               lo=None):
    share = n_edges // nsplit
    nwin = share // WIN

    def body(w, _):
        base = wid * share + w * WIN
        pltpu.sync_copy(src_hbm.at[pl.ds(base, WIN)], srcb)
        pltpu.sync_copy(dst_hbm.at[pl.ds(base, WIN)], dstb)
        pltpu.sync_copy(w_hbm.at[pl.ds(base, WIN)], wb)
        gsrc = wb
        if lo is not None:
            for i in range(WIN // 16):
                d16 = dstb[pl.ds(16 * i, 16)]
                w16 = wb[pl.ds(16 * i, 16)]
                inr = (d16 >= lo) & (d16 < lo + WRANGE)
                gb[pl.ds(16 * i, 16)] = jnp.where(inr, w16, 0.0)
                dstb[pl.ds(16 * i, 16)] = jnp.clip(d16 - lo, 0, WRANGE - 1)
            gsrc = gb
        pltpu.async_copy(h_hbm.at[srcb], rows, sem).wait()

        def scale(i, _):
            g16 = gsrc[pl.ds(16 * i, 16)]
            for j in range(16):
                e = 16 * i + j
                g = g16[j]
                for c in range(HD // 16):
                    rows[e, pl.ds(16 * c, 16)] = rows[e, pl.ds(16 * c, 16)] * g
            return 0
        lax.fori_loop(0, WIN // 16, scale, 0)
        pltpu.sync_copy(rows, acc_sh.at[dstb], add=True)
        return 0
    lax.fori_loop(0, nwin, body, 0)


def _pieces(total):
    out, off = [], 0
    while off < total:
        n = min(128, total - off)
        assert n % 16 == 0
        out.append((off, n))
        off += n
    return out


def _scaled_emit(acc_sh, cnt_sh, rows, cbuf, row0, n, out_slice):
    pltpu.sync_copy(acc_sh.at[pl.ds(row0, n)], rows.at[pl.ds(0, n)])
    pltpu.sync_copy(cnt_sh.at[pl.ds(row0, n)], cbuf.at[pl.ds(0, n)])

    def grp(i, _):
        c16 = cbuf[pl.ds(16 * i, 16)]
        r16 = 1.0 / jnp.maximum(c16, 1.0)
        for j in range(16):
            e = 16 * i + j
            r = r16[j]
            for c in range(HD // 16):
                rows[e, pl.ds(16 * c, 16)] = rows[e, pl.ds(16 * c, 16)] * r
        return 0
    lax.fori_loop(0, n // 16, grp, 0)
    pltpu.sync_copy(rows.at[pl.ds(0, n)], out_slice)


@functools.partial(
    pl.kernel, mesh=_mesh,
    compiler_params=pltpu.CompilerParams(needs_layout_passes=False),
    out_type=jax.ShapeDtypeStruct((GROWS, HD), jnp.float32),
    scratch_types=[
        pltpu.VMEM((WIN,), jnp.int32),
        pltpu.VMEM((WIN, HD), jnp.float32),
        pltpu.SemaphoreType.DMA,
    ],
)
def _gather_embeds(table_hbm, ids_hbm, out_hbm, idxb, rowsb, sem):
    cid = lax.axis_index("c")
    sid = lax.axis_index("s")
    wid = sid * 2 + cid
    share = GROWS // 32

    def body(w, _):
        base = wid * share + w * WIN
        pltpu.sync_copy(ids_hbm.at[pl.ds(base, WIN)], idxb)
        pltpu.async_copy(table_hbm.at[idxb], rowsb, sem).wait()
        pltpu.sync_copy(rowsb, out_hbm.at[pl.ds(base, WIN)])
        return 0
    lax.fori_loop(0, share // WIN, body, 0)


def _make_pair_agg(n_e1, n_e2, cacc):
    zlen = cacc // 16

    @functools.partial(
        pl.kernel, mesh=_mesh,
        compiler_params=pltpu.CompilerParams(needs_layout_passes=False),
        out_type=jax.ShapeDtypeStruct((4, cacc, HD), jnp.float32),
        scratch_types=[
            pltpu.VMEM_SHARED((cacc, HD), jnp.float32),
            pltpu.VMEM_SHARED((cacc, HD), jnp.float32),
            pltpu.VMEM_SHARED((cacc,), jnp.float32),
            pltpu.VMEM_SHARED((cacc,), jnp.float32),
            pltpu.VMEM((zlen,), jnp.float32),
            pltpu.VMEM((WIN,), jnp.float32),
            pltpu.VMEM((WIN,), jnp.int32),
            pltpu.VMEM((WIN,), jnp.int32),
            pltpu.VMEM((WIN,), jnp.float32),
            pltpu.VMEM((WIN,), jnp.float32),
            pltpu.VMEM((WIN, HD), jnp.float32),
            pltpu.SemaphoreType.DMA,
        ],
    )
    def k(s1, d1, w1, ha, s2, d2, w2, hb, out_hbm,
          accA, accB, cntA, cntB, z1, ones,
          srcb, dstb, wb, cbuf, rows, sem):
        cid = lax.axis_index("c")
        sid = lax.axis_index("s")
        wid = sid * 2 + cid
        _fill_zero_2d(rows, WIN)
        _fill_zero_1d(z1, zlen)
        _fill_ones_1d(ones, WIN)
        chunk = cacc // 16
        _zero_shared_rows(accA, rows, sid * chunk, chunk)
        _zero_shared_rows(accB, rows, sid * chunk, chunk)
        pltpu.sync_copy(z1, cntA.at[pl.ds(sid * zlen, zlen)])
        pltpu.sync_copy(z1, cntB.at[pl.ds(sid * zlen, zlen)])
        plsc.subcore_barrier()
        _count_pass(d1, cntA, ones, dstb, n_e1, sid)
        _count_pass(d2, cntB, ones, dstb, n_e2, sid)
        _edge_pass(s1, d1, w1, ha, accA, srcb, dstb, wb, None, rows, sem,
                   n_e1, wid, 32)
        _edge_pass(s2, d2, w2, hb, accB, srcb, dstb, wb, None, rows, sem,
                   n_e2, wid, 32)
        plsc.subcore_barrier()
        for off, n in _pieces(chunk):
            row0 = sid * chunk + off
            _scaled_emit(accA, cntA, rows, cbuf, row0, n,
                         out_hbm.at[cid, pl.ds(row0, n)])
            _scaled_emit(accB, cntB, rows, cbuf, row0, n,
                         out_hbm.at[2 + cid, pl.ds(row0, n)])
    return k


_agg_topic = _make_pair_agg(WT_E, TT_E, CT)
_agg_doc = _make_pair_agg(WD_E, TD_E, CD)


@functools.partial(
    pl.kernel, mesh=_mesh,
    compiler_params=pltpu.CompilerParams(needs_layout_passes=False),
    out_type=jax.ShapeDtypeStruct((CW, HD), jnp.float32),
    scratch_types=[
        pltpu.VMEM_SHARED((WRANGE, HD), jnp.float32),
        pltpu.VMEM_SHARED((CW,), jnp.float32),
        pltpu.VMEM((CW // 16,), jnp.float32),
        pltpu.VMEM((WIN,), jnp.float32),
        pltpu.VMEM((WIN,), jnp.int32),
        pltpu.VMEM((WIN,), jnp.int32),
        pltpu.VMEM((WIN,), jnp.float32),
        pltpu.VMEM((WIN,), jnp.float32),
        pltpu.VMEM((WIN,), jnp.float32),
        pltpu.VMEM((WIN, HD), jnp.float32),
        pltpu.SemaphoreType.DMA,
    ],
)
def _agg_ww(src_hbm, dst_hbm, w_hbm, h_hbm, out_hbm,
            acc_sh, cnt_sh, z1, ones,
            srcb, dstb, wb, gb, cbuf, rows, sem):
    cid = lax.axis_index("c")
    sid = lax.axis_index("s")
    _fill_zero_1d(z1, CW // 16)
    _fill_ones_1d(ones, WIN)
    pltpu.sync_copy(z1, cnt_sh.at[pl.ds(sid * (CW // 16), CW // 16)])
    plsc.subcore_barrier()
    _count_pass(dst_hbm, cnt_sh, ones, dstb, WW_E, sid)
    chunk = WRANGE // 16
    for p in range(2):
        lo = (cid * 2 + p) * WRANGE
        _fill_zero_2d(rows, WIN)
        _zero_shared_rows(acc_sh, rows, sid * chunk, chunk)
        plsc.subcore_barrier()
        _edge_pass(src_hbm, dst_hbm, w_hbm, h_hbm, acc_sh,
                   srcb, dstb, wb, gb, rows, sem, WW_E, sid, 16, lo=lo)
        plsc.subcore_barrier()
        for off, n in _pieces(chunk):
            _scaled_emit(acc_sh, cnt_sh.at[pl.ds(lo, WRANGE)], rows, cbuf,
                         sid * chunk + off, n,
                         out_hbm.at[pl.ds(lo + sid * chunk + off, n)])
        plsc.subcore_barrier()


def _prep_body(aW, ab, cW, cb, tids, temb,
               m0_o, b0_o, m1_o, b1_o, t1_o, tb1_o, ht0_o):
    m0 = aW[...]
    b0 = ab[...]
    for i in range(3):
        w = cW[0, i]
        m0 = jnp.dot(m0, w, preferred_element_type=jnp.float32)
        b0 = jnp.dot(b0, w, preferred_element_type=jnp.float32) + cb[0, i]
    m1 = cW[1, 0]
    b1 = cb[1, 0]
    for i in range(1, 3):
        w = cW[1, i]
        m1 = jnp.dot(m1, w, preferred_element_type=jnp.float32)
        b1 = jnp.dot(b1, w, preferred_element_type=jnp.float32) + cb[1, i]
    t1 = jnp.dot(cW[1, 3], cW[1, 4], preferred_element_type=jnp.float32)
    tb1 = jnp.dot(cb[1, 3], cW[1, 4], preferred_element_type=jnp.float32) + cb[1, 4]
    tt0 = jnp.dot(temb[...], cW[0, 3], preferred_element_type=jnp.float32) + cb[0, 3]
    tt0 = jnp.dot(tt0, cW[0, 4], preferred_element_type=jnp.float32) + cb[0, 4]
    ids = tids[...]
    oh = (lax.broadcasted_iota(jnp.int32, (NTN, 64), 1) == ids).astype(jnp.float32)
    ht0_o[...] = jnp.dot(oh, tt0, preferred_element_type=jnp.float32)
    m0_o[...] = m0
    b0_o[...] = b0
    m1_o[...] = m1
    b1_o[...] = b1
    t1_o[...] = t1
    tb1_o[...] = tb1


def _prep(aW, ab, cW, cb, tids, temb):
    return pl.pallas_call(
        _prep_body,
        out_shape=(
            jax.ShapeDtypeStruct((HI, HD), jnp.float32),
            jax.ShapeDtypeStruct((1, HD), jnp.float32),
            jax.ShapeDtypeStruct((HD, HD), jnp.float32),
            jax.ShapeDtypeStruct((1, HD), jnp.float32),
            jax.ShapeDtypeStruct((HD, HD), jnp.float32),
            jax.ShapeDtypeStruct((1, HD), jnp.float32),
            jax.ShapeDtypeStruct((NTN, HD), jnp.float32),
        ),
    )(aW, ab, cW, cb, tids, temb)


def _matmul_bias(x, m, b, relu_in=False, tr=512):
    n, k = x.shape
    assert n % tr == 0

    def body(x_ref, m_ref, b_ref, o_ref):
        xv = x_ref[...]
        if relu_in:
            xv = jnp.maximum(xv, 0.0)
        o_ref[...] = jnp.dot(xv, m_ref[...], preferred_element_type=jnp.float32) + b_ref[...]

    return pl.pallas_call(
        body,
        grid=(n // tr,),
        in_specs=[
            pl.BlockSpec((tr, k), lambda i: (i, 0)),
            pl.BlockSpec((k, HD), lambda i: (0, 0)),
            pl.BlockSpec((1, HD), lambda i: (0, 0)),
        ],
        out_specs=pl.BlockSpec((tr, HD), lambda i: (i, 0)),
        out_shape=jax.ShapeDtypeStruct((n, HD), jnp.float32),
    )(x, m, b)


def _combine_pair_matmul(acc, m, b):
    _, n, _ = acc.shape

    def body(a_ref, m_ref, b_ref, o_ref):
        h = jnp.maximum(a_ref[0] + a_ref[1] + a_ref[2] + a_ref[3], 0.0)
        o_ref[...] = jnp.dot(h, m_ref[...], preferred_element_type=jnp.float32) + b_ref[...]

    return pl.pallas_call(
        body,
        out_shape=jax.ShapeDtypeStruct((n, HD), jnp.float32),
    )(acc, m, b)


def _final(acc, ow, ob, y):
    def body(a_ref, ow_ref, ob_ref, y_ref, loss_ref, yp_ref):
        ms = []
        for g in range(NB):
            blk = (a_ref[0, pl.ds(100 * g, 100), :] + a_ref[1, pl.ds(100 * g, 100), :]
                   + a_ref[2, pl.ds(100 * g, 100), :] + a_ref[3, pl.ds(100 * g, 100), :])
            blk = jnp.maximum(blk, 0.0)
            ms.append(jnp.max(blk, axis=0, keepdims=True))
        glob = jnp.concatenate(ms, axis=0)
        z = jnp.sum(glob * ow_ref[...], axis=1, keepdims=True) + ob_ref[...]
        yv = y_ref[...]
        lossv = jnp.mean(jnp.maximum(z, 0.0) - z * yv
                         + jnp.log(1.0 + jnp.exp(-jnp.abs(z))))
        loss_ref[...] = lossv[None, None]
        yp_ref[...] = 1.0 / (1.0 + jnp.exp(-z))

    return pl.pallas_call(
        body,
        out_shape=(
            jax.ShapeDtypeStruct((1, 1), jnp.float32),
            jax.ShapeDtypeStruct((NB, 1), jnp.float32),
        ),
    )(acc, ow, ob, y)


def _pad_edges(src, dst, w, n_pad, n_src, n_dst):
    e = src.shape[0]
    k = n_pad - e
    pad_src = (jnp.arange(k, dtype=jnp.int32) * 7919) % n_src
    src = jnp.concatenate([src.astype(jnp.int32), pad_src])
    dst = jnp.concatenate([dst.astype(jnp.int32),
                           jnp.full((k,), n_dst, jnp.int32)])
    w = jnp.concatenate([w, jnp.zeros((k,), w.dtype)])
    return src, dst, w


def kernel(word_ids, topic_ids, ww_src, ww_dst, ww_w, wt_src, wt_dst, wt_w,
           wd_src, wd_dst, wd_w, td_src, td_dst, td_w, tt_src, tt_dst, tt_w,
           doc_graph_ids, y_data, word_embeds, topic_embeds, adapt_W, adapt_b,
           conv_W, conv_b, out_W, out_b):
    wid_pad = jnp.concatenate([
        word_ids.astype(jnp.int32),
        (jnp.arange(GROWS - NWN, dtype=jnp.int32) * 7919) % VOC])
    ww = _pad_edges(ww_src, ww_dst, ww_w, WW_E, NWN, NWN)
    wt = _pad_edges(wt_src, wt_dst, wt_w, WT_E, NWN, NTN)
    wd = _pad_edges(wd_src, wd_dst, wd_w, WD_E, NWN, NDN)
    td = _pad_edges(td_src, td_dst, td_w, TD_E, NTN, NDN)
    tt = _pad_edges(tt_src, tt_dst, tt_w, TT_E, NTN, NTN)
    temb_pad = jnp.concatenate(
        [topic_embeds, jnp.zeros((14, HD), jnp.float32)], axis=0)
    cb4 = conv_b.reshape(2, 5, 1, HD)
    ab2 = adapt_b.reshape(1, HD)
    tids2 = topic_ids.astype(jnp.int32).reshape(NTN, 1)
    y2 = y_data.reshape(NB, 1)
    ow2 = out_W.reshape(1, HD)
    ob2 = out_b.reshape(1, 1)

    m0, b0, m1, b1, t1, tb1, ht0 = _prep(adapt_W, ab2, conv_W, cb4, tids2, temb_pad)

    tword = _matmul_bias(word_embeds, m0, b0, tr=1000)
    hw0 = _gather_embeds(tword, wid_pad)

    at = _agg_topic(wt[0], wt[1], wt[2], hw0, tt[0], tt[1], tt[2], ht0)
    aww = _agg_ww(ww[0], ww[1], ww[2], hw0)

    hw1 = _matmul_bias(aww, m1, b1, relu_in=True)
    ht1 = _combine_pair_matmul(at, t1, tb1)

    ad = _agg_doc(wd[0], wd[1], wd[2], hw1, td[0], td[1], td[2], ht1)

    loss2, yp = _final(ad, ow2, ob2, y2)
    return (loss2.reshape(()), yp)


# speedup vs baseline: 2.9506x; 1.6718x over previous
import functools

import jax
import jax.numpy as jnp
from jax import lax
from jax.experimental import pallas as pl
from jax.experimental.pallas import tpu as pltpu
from jax.experimental.pallas import tpu_sc as plsc

HD = 128
NWN = 50000
NTN = 3200
NDN = 6400
NB = 64
VOC = 100000
HI = 300

WW_E, WT_E, WD_E, TD_E, TT_E = 323584, 81920, 163840, 40960, 20480
GROWS = 53248
CW = 50176
WRANGE = 12544
CT = 3328
CD = 6656
WIN = 128

_mesh = plsc.VectorSubcoreMesh(core_axis_name="c", subcore_axis_name="s")


def _fill_zero_2d(ref, nrows):
    def body(i, _):
        for c in range(HD // 16):
            ref[i, pl.ds(16 * c, 16)] = jnp.zeros((16,), jnp.float32)
        return 0
    lax.fori_loop(0, nrows, body, 0)


def _fill_zero_1d(ref, n):
    def body(i, _):
        ref[pl.ds(16 * i, 16)] = jnp.zeros((16,), jnp.float32)
        return 0
    lax.fori_loop(0, n // 16, body, 0)


def _fill_ones_1d(ref, n):
    for i in range(n // 16):
        ref[pl.ds(16 * i, 16)] = jnp.ones((16,), jnp.float32)


def _zero_shared_rows(acc_sh, zbuf, row0, nrows):
    off = 0
    while off < nrows:
        n = min(128, nrows - off)
        pltpu.sync_copy(zbuf.at[pl.ds(0, n)], acc_sh.at[pl.ds(row0 + off, n)])
        off += n


def _count_pass(dst_hbm, cnt_sh, ones, dstb, n_edges, sid):
    share = n_edges // 16
    nwin = share // WIN

    def body(w, _):
        base = sid * share + w * WIN
        pltpu.sync_copy(dst_hbm.at[pl.ds(base, WIN)], dstb)
        pltpu.sync_copy(ones, cnt_sh.at[dstb], add=True)
        return 0
    lax.fori_loop(0, nwin, body, 0)


def _edge_pass(src_hbm, dst_hbm, w_hbm, h_hbm, acc_sh,
               srcb, dstb, wb, gb, rows, sem, n_edges, wid, nsplit,
               lo=None):
    share = n_edges // nsplit
    nwin = share // WIN

    def body(w, _):
        base = wid * share + w * WIN
        pltpu.sync_copy(src_hbm.at[pl.ds(base, WIN)], srcb)
        pltpu.sync_copy(dst_hbm.at[pl.ds(base, WIN)], dstb)
        pltpu.sync_copy(w_hbm.at[pl.ds(base, WIN)], wb)
        gsrc = wb
        if lo is not None:
            for i in range(WIN // 16):
                d16 = dstb[pl.ds(16 * i, 16)]
                w16 = wb[pl.ds(16 * i, 16)]
                inr = (d16 >= lo) & (d16 < lo + WRANGE)
                gb[pl.ds(16 * i, 16)] = jnp.where(inr, w16, 0.0)
                dstb[pl.ds(16 * i, 16)] = jnp.clip(d16 - lo, 0, WRANGE - 1)
            gsrc = gb
        pltpu.async_copy(h_hbm.at[srcb], rows, sem).wait()

        def scale(i, _):
            g16 = gsrc[pl.ds(16 * i, 16)]
            for j in range(16):
                e = 16 * i + j
                g = g16[j]
                for c in range(HD // 16):
                    rows[e, pl.ds(16 * c, 16)] = rows[e, pl.ds(16 * c, 16)] * g
            return 0
        lax.fori_loop(0, WIN // 16, scale, 0)
        pltpu.sync_copy(rows, acc_sh.at[dstb], add=True)
        return 0
    lax.fori_loop(0, nwin, body, 0)


def _pieces(total):
    out, off = [], 0
    while off < total:
        n = min(128, total - off)
        assert n % 16 == 0
        out.append((off, n))
        off += n
    return out


def _scaled_emit(acc_sh, cnt_sh, rows, cbuf, row0, n, out_slice):
    pltpu.sync_copy(acc_sh.at[pl.ds(row0, n)], rows.at[pl.ds(0, n)])
    pltpu.sync_copy(cnt_sh.at[pl.ds(row0, n)], cbuf.at[pl.ds(0, n)])

    def grp(i, _):
        c16 = cbuf[pl.ds(16 * i, 16)]
        r16 = 1.0 / jnp.maximum(c16, 1.0)
        for j in range(16):
            e = 16 * i + j
            r = r16[j]
            for c in range(HD // 16):
                rows[e, pl.ds(16 * c, 16)] = rows[e, pl.ds(16 * c, 16)] * r
        return 0
    lax.fori_loop(0, n // 16, grp, 0)
    pltpu.sync_copy(rows.at[pl.ds(0, n)], out_slice)


@functools.partial(
    pl.kernel, mesh=_mesh,
    compiler_params=pltpu.CompilerParams(needs_layout_passes=False),
    out_type=jax.ShapeDtypeStruct((GROWS, HD), jnp.float32),
    scratch_types=[
        pltpu.VMEM((WIN,), jnp.int32),
        pltpu.VMEM((WIN, HD), jnp.float32),
        pltpu.SemaphoreType.DMA,
    ],
)
def _gather_embeds(table_hbm, ids_hbm, out_hbm, idxb, rowsb, sem):
    cid = lax.axis_index("c")
    sid = lax.axis_index("s")
    wid = sid * 2 + cid
    share = GROWS // 32

    def body(w, _):
        base = wid * share + w * WIN
        pltpu.sync_copy(ids_hbm.at[pl.ds(base, WIN)], idxb)
        pltpu.async_copy(table_hbm.at[idxb], rowsb, sem).wait()
        pltpu.sync_copy(rowsb, out_hbm.at[pl.ds(base, WIN)])
        return 0
    lax.fori_loop(0, share // WIN, body, 0)


def _make_pair_agg(n_e1, n_e2, cacc):
    zlen = cacc // 16

    @functools.partial(
        pl.kernel, mesh=_mesh,
        compiler_params=pltpu.CompilerParams(needs_layout_passes=False),
        out_type=jax.ShapeDtypeStruct((4, cacc, HD), jnp.float32),
        scratch_types=[
            pltpu.VMEM_SHARED((cacc, HD), jnp.float32),
            pltpu.VMEM_SHARED((cacc, HD), jnp.float32),
            pltpu.VMEM_SHARED((cacc,), jnp.float32),
            pltpu.VMEM_SHARED((cacc,), jnp.float32),
            pltpu.VMEM((zlen,), jnp.float32),
            pltpu.VMEM((WIN,), jnp.float32),
            pltpu.VMEM((WIN,), jnp.int32),
            pltpu.VMEM((WIN,), jnp.int32),
            pltpu.VMEM((WIN,), jnp.float32),
            pltpu.VMEM((WIN,), jnp.float32),
            pltpu.VMEM((WIN, HD), jnp.float32),
            pltpu.SemaphoreType.DMA,
        ],
    )
    def k(s1, d1, w1, ha, s2, d2, w2, hb, out_hbm,
          accA, accB, cntA, cntB, z1, ones,
          srcb, dstb, wb, cbuf, rows, sem):
        cid = lax.axis_index("c")
        sid = lax.axis_index("s")
        wid = sid * 2 + cid
        _fill_zero_2d(rows, WIN)
        _fill_zero_1d(z1, zlen)
        _fill_ones_1d(ones, WIN)
        chunk = cacc // 16
        _zero_shared_rows(accA, rows, sid * chunk, chunk)
        _zero_shared_rows(accB, rows, sid * chunk, chunk)
        pltpu.sync_copy(z1, cntA.at[pl.ds(sid * zlen, zlen)])
        pltpu.sync_copy(z1, cntB.at[pl.ds(sid * zlen, zlen)])
        plsc.subcore_barrier()
        _count_pass(d1, cntA, ones, dstb, n_e1, sid)
        _count_pass(d2, cntB, ones, dstb, n_e2, sid)
        _edge_pass(s1, d1, w1, ha, accA, srcb, dstb, wb, None, rows, sem,
                   n_e1, wid, 32)
        _edge_pass(s2, d2, w2, hb, accB, srcb, dstb, wb, None, rows, sem,
                   n_e2, wid, 32)
        plsc.subcore_barrier()
        for off, n in _pieces(chunk):
            row0 = sid * chunk + off
            _scaled_emit(accA, cntA, rows, cbuf, row0, n,
                         out_hbm.at[cid, pl.ds(row0, n)])
            _scaled_emit(accB, cntB, rows, cbuf, row0, n,
                         out_hbm.at[2 + cid, pl.ds(row0, n)])
    return k


_agg_topic = _make_pair_agg(WT_E, TT_E, CT)
_agg_doc = _make_pair_agg(WD_E, TD_E, CD)


WW_SHARE = WW_E // 16
WW_SCHUNK = 1264
WW_CBUF = WW_SCHUNK + 144


@functools.partial(
    pl.kernel, mesh=_mesh,
    compiler_params=pltpu.CompilerParams(needs_layout_passes=False),
    out_type=jax.ShapeDtypeStruct((CW, HD), jnp.float32),
    scratch_types=[
        pltpu.VMEM_SHARED((WRANGE, HD), jnp.float32),
        pltpu.VMEM_SHARED((CW,), jnp.float32),
        pltpu.VMEM((WW_SCHUNK,), jnp.int32),
        pltpu.VMEM((WW_SCHUNK,), jnp.int32),
        pltpu.VMEM((WW_SCHUNK,), jnp.float32),
        pltpu.VMEM((WW_CBUF,), jnp.int32),
        pltpu.VMEM((WW_CBUF,), jnp.int32),
        pltpu.VMEM((WW_CBUF,), jnp.float32),
        pltpu.VMEM((WIN,), jnp.float32),
        pltpu.VMEM((WIN,), jnp.int32),
        pltpu.VMEM((WIN,), jnp.int32),
        pltpu.VMEM((WIN,), jnp.float32),
        pltpu.VMEM((WIN,), jnp.float32),
        pltpu.VMEM((WIN, HD), jnp.float32),
        pltpu.SemaphoreType.DMA,
    ],
)
def _agg_ww(src_hbm, dst_hbm, w_hbm, h_hbm, out_hbm,
            acc_sh, cnt_sh, srcsc, dstsc, wsc, csrc, cdloc, cgain,
            ones, srcb, dstb, gb, cbuf, rows, sem):
    cid = lax.axis_index("c")
    sid = lax.axis_index("s")
    wid = sid * 2 + cid
    _fill_ones_1d(ones, WIN)
    _fill_zero_1d(cgain, WW_CBUF)
    zc = CW // 16
    zoff = 0
    while zoff < zc:
        n = min(WW_CBUF - (WW_CBUF % 8), zc - zoff)
        n = n - (n % 8)
        pltpu.sync_copy(cgain.at[pl.ds(0, n)],
                        cnt_sh.at[pl.ds(sid * zc + zoff, n)])
        zoff += n
    plsc.subcore_barrier()
    ncwin = WW_SHARE // WIN

    def cbody(w, _):
        base = sid * WW_SHARE + w * WIN
        pltpu.sync_copy(dst_hbm.at[pl.ds(base, WIN)], dstb)
        pltpu.sync_copy(ones, cnt_sh.at[dstb], add=True)
        return 0
    lax.fori_loop(0, ncwin, cbody, 0)
    chunk = WRANGE // 16
    iota16 = lax.broadcasted_iota(jnp.int32, (16,), 0)
    for p in range(2):
        lo = (cid * 2 + p) * WRANGE
        _fill_zero_2d(rows, WIN)
        _zero_shared_rows(acc_sh, rows, sid * chunk, chunk)
        plsc.subcore_barrier()

        def schunk(scn, _):
            base = sid * WW_SHARE + scn * WW_SCHUNK
            pltpu.sync_copy(src_hbm.at[pl.ds(base, WW_SCHUNK)], srcsc)
            pltpu.sync_copy(dst_hbm.at[pl.ds(base, WW_SCHUNK)], dstsc)
            pltpu.sync_copy(w_hbm.at[pl.ds(base, WW_SCHUNK)], wsc)

            def compact(i, pos):
                d16 = dstsc[pl.ds(16 * i, 16)]
                w16 = wsc[pl.ds(16 * i, 16)]
                s16 = srcsc[pl.ds(16 * i, 16)]
                inr = (d16 >= lo) & (d16 < lo + WRANGE)
                plsc.store_compressed(csrc.at[pl.ds(pos, 16)], s16, mask=inr)
                plsc.store_compressed(cdloc.at[pl.ds(pos, 16)], d16 - lo, mask=inr)
                plsc.store_compressed(cgain.at[pl.ds(pos, 16)], w16, mask=inr)
                return pos + plsc.all_reduce_population_count(inr)[0]
            pos = lax.fori_loop(0, WW_SCHUNK // 16, compact, jnp.int32(0))
            for k in range(8):
                pad_idx = iota16 + (16 * k + wid * 128)
                csrc[pl.ds(pos + 16 * k, 16)] = pad_idx
                cdloc[pl.ds(pos + 16 * k, 16)] = jnp.zeros((16,), jnp.int32)
                cgain[pl.ds(pos + 16 * k, 16)] = jnp.zeros((16,), jnp.float32)

            def cwin(k, _):
                for g in range(8):
                    srcb[pl.ds(16 * g, 16)] = csrc[pl.ds(128 * k + 16 * g, 16)]
                    dstb[pl.ds(16 * g, 16)] = cdloc[pl.ds(128 * k + 16 * g, 16)]
                    gb[pl.ds(16 * g, 16)] = cgain[pl.ds(128 * k + 16 * g, 16)]
                pltpu.async_copy(h_hbm.at[srcb], rows, sem).wait()

                def scale(i, _):
                    g16 = gb[pl.ds(16 * i, 16)]
                    for j in range(16):
                        e = 16 * i + j
                        g = g16[j]
                        for c in range(HD // 16):
                            rows[e, pl.ds(16 * c, 16)] = rows[e, pl.ds(16 * c, 16)] * g
                    return 0
                lax.fori_loop(0, WIN // 16, scale, 0)
                pltpu.sync_copy(rows, acc_sh.at[dstb], add=True)
                return 0
            lax.fori_loop(0, (pos + 127) // 128, cwin, 0)
            return 0
        lax.fori_loop(0, WW_SHARE // WW_SCHUNK, schunk, 0)
        plsc.subcore_barrier()
        for off, n in _pieces(chunk):
            _scaled_emit(acc_sh, cnt_sh.at[pl.ds(lo, WRANGE)], rows, cbuf,
                         sid * chunk + off, n,
                         out_hbm.at[pl.ds(lo + sid * chunk + off, n)])
        plsc.subcore_barrier()


def _prep_body(aW, ab, cW, cb, tids, temb,
               m0_o, b0_o, m1_o, b1_o, t1_o, tb1_o, ht0_o):
    m0 = aW[...]
    b0 = ab[...]
    for i in range(3):
        w = cW[0, i]
        m0 = jnp.dot(m0, w, preferred_element_type=jnp.float32)
        b0 = jnp.dot(b0, w, preferred_element_type=jnp.float32) + cb[0, i]
    m1 = cW[1, 0]
    b1 = cb[1, 0]
    for i in range(1, 3):
        w = cW[1, i]
        m1 = jnp.dot(m1, w, preferred_element_type=jnp.float32)
        b1 = jnp.dot(b1, w, preferred_element_type=jnp.float32) + cb[1, i]
    t1 = jnp.dot(cW[1, 3], cW[1, 4], preferred_element_type=jnp.float32)
    tb1 = jnp.dot(cb[1, 3], cW[1, 4], preferred_element_type=jnp.float32) + cb[1, 4]
    tt0 = jnp.dot(temb[...], cW[0, 3], preferred_element_type=jnp.float32) + cb[0, 3]
    tt0 = jnp.dot(tt0, cW[0, 4], preferred_element_type=jnp.float32) + cb[0, 4]
    ids = tids[...]
    oh = (lax.broadcasted_iota(jnp.int32, (NTN, 64), 1) == ids).astype(jnp.float32)
    ht0_o[...] = jnp.dot(oh, tt0, preferred_element_type=jnp.float32)
    m0_o[...] = m0
    b0_o[...] = b0
    m1_o[...] = m1
    b1_o[...] = b1
    t1_o[...] = t1
    tb1_o[...] = tb1


def _prep(aW, ab, cW, cb, tids, temb):
    return pl.pallas_call(
        _prep_body,
        out_shape=(
            jax.ShapeDtypeStruct((HI, HD), jnp.float32),
            jax.ShapeDtypeStruct((1, HD), jnp.float32),
            jax.ShapeDtypeStruct((HD, HD), jnp.float32),
            jax.ShapeDtypeStruct((1, HD), jnp.float32),
            jax.ShapeDtypeStruct((HD, HD), jnp.float32),
            jax.ShapeDtypeStruct((1, HD), jnp.float32),
            jax.ShapeDtypeStruct((NTN, HD), jnp.float32),
        ),
    )(aW, ab, cW, cb, tids, temb)


def _matmul_bias(x, m, b, relu_in=False, tr=512):
    n, k = x.shape
    assert n % tr == 0

    def body(x_ref, m_ref, b_ref, o_ref):
        xv = x_ref[...]
        if relu_in:
            xv = jnp.maximum(xv, 0.0)
        o_ref[...] = jnp.dot(xv, m_ref[...], preferred_element_type=jnp.float32) + b_ref[...]

    return pl.pallas_call(
        body,
        grid=(n // tr,),
        in_specs=[
            pl.BlockSpec((tr, k), lambda i: (i, 0)),
            pl.BlockSpec((k, HD), lambda i: (0, 0)),
            pl.BlockSpec((1, HD), lambda i: (0, 0)),
        ],
        out_specs=pl.BlockSpec((tr, HD), lambda i: (i, 0)),
        out_shape=jax.ShapeDtypeStruct((n, HD), jnp.float32),
    )(x, m, b)


def _combine_pair_matmul(acc, m, b):
    _, n, _ = acc.shape

    def body(a_ref, m_ref, b_ref, o_ref):
        h = jnp.maximum(a_ref[0] + a_ref[1] + a_ref[2] + a_ref[3], 0.0)
        o_ref[...] = jnp.dot(h, m_ref[...], preferred_element_type=jnp.float32) + b_ref[...]

    return pl.pallas_call(
        body,
        out_shape=jax.ShapeDtypeStruct((n, HD), jnp.float32),
    )(acc, m, b)


def _final(acc, ow, ob, y):
    def body(a_ref, ow_ref, ob_ref, y_ref, loss_ref, yp_ref):
        ms = []
        for g in range(NB):
            blk = (a_ref[0, pl.ds(100 * g, 100), :] + a_ref[1, pl.ds(100 * g, 100), :]
                   + a_ref[2, pl.ds(100 * g, 100), :] + a_ref[3, pl.ds(100 * g, 100), :])
            blk = jnp.maximum(blk, 0.0)
            ms.append(jnp.max(blk, axis=0, keepdims=True))
        glob = jnp.concatenate(ms, axis=0)
        z = jnp.sum(glob * ow_ref[...], axis=1, keepdims=True) + ob_ref[...]
        yv = y_ref[...]
        lossv = jnp.mean(jnp.maximum(z, 0.0) - z * yv
                         + jnp.log(1.0 + jnp.exp(-jnp.abs(z))))
        loss_ref[...] = lossv[None, None]
        yp_ref[...] = 1.0 / (1.0 + jnp.exp(-z))

    return pl.pallas_call(
        body,
        out_shape=(
            jax.ShapeDtypeStruct((1, 1), jnp.float32),
            jax.ShapeDtypeStruct((NB, 1), jnp.float32),
        ),
    )(acc, ow, ob, y)


def _pad_edges(src, dst, w, n_pad, n_src, n_dst):
    e = src.shape[0]
    k = n_pad - e
    pad_src = (jnp.arange(k, dtype=jnp.int32) * 7919) % n_src
    src = jnp.concatenate([src.astype(jnp.int32), pad_src])
    dst = jnp.concatenate([dst.astype(jnp.int32),
                           jnp.full((k,), n_dst, jnp.int32)])
    w = jnp.concatenate([w, jnp.zeros((k,), w.dtype)])
    return src, dst, w


def kernel(word_ids, topic_ids, ww_src, ww_dst, ww_w, wt_src, wt_dst, wt_w,
           wd_src, wd_dst, wd_w, td_src, td_dst, td_w, tt_src, tt_dst, tt_w,
           doc_graph_ids, y_data, word_embeds, topic_embeds, adapt_W, adapt_b,
           conv_W, conv_b, out_W, out_b):
    wid_pad = jnp.concatenate([
        word_ids.astype(jnp.int32),
        (jnp.arange(GROWS - NWN, dtype=jnp.int32) * 7919) % VOC])
    ww = _pad_edges(ww_src, ww_dst, ww_w, WW_E, NWN, NWN)
    wt = _pad_edges(wt_src, wt_dst, wt_w, WT_E, NWN, NTN)
    wd = _pad_edges(wd_src, wd_dst, wd_w, WD_E, NWN, NDN)
    td = _pad_edges(td_src, td_dst, td_w, TD_E, NTN, NDN)
    tt = _pad_edges(tt_src, tt_dst, tt_w, TT_E, NTN, NTN)
    temb_pad = jnp.concatenate(
        [topic_embeds, jnp.zeros((14, HD), jnp.float32)], axis=0)
    cb4 = conv_b.reshape(2, 5, 1, HD)
    ab2 = adapt_b.reshape(1, HD)
    tids2 = topic_ids.astype(jnp.int32).reshape(NTN, 1)
    y2 = y_data.reshape(NB, 1)
    ow2 = out_W.reshape(1, HD)
    ob2 = out_b.reshape(1, 1)

    m0, b0, m1, b1, t1, tb1, ht0 = _prep(adapt_W, ab2, conv_W, cb4, tids2, temb_pad)

    tword = _matmul_bias(word_embeds, m0, b0, tr=1000)
    hw0 = _gather_embeds(tword, wid_pad)

    at = _agg_topic(wt[0], wt[1], wt[2], hw0, tt[0], tt[1], tt[2], ht0)
    aww = _agg_ww(ww[0], ww[1], ww[2], hw0)

    hw1 = _matmul_bias(aww, m1, b1, relu_in=True)
    ht1 = _combine_pair_matmul(at, t1, tb1)

    ad = _agg_doc(wd[0], wd[1], wd[2], hw1, td[0], td[1], td[2], ht1)

    loss2, yp = _final(ad, ow2, ob2, y2)
    return (loss2.reshape(()), yp)


# speedup vs baseline: 3.6908x; 1.2509x over previous
import functools

import jax
import jax.numpy as jnp
from jax import lax
from jax.experimental import pallas as pl
from jax.experimental.pallas import tpu as pltpu
from jax.experimental.pallas import tpu_sc as plsc

HD = 128
NWN = 50000
NTN = 3200
NDN = 6400
NB = 64
VOC = 100000
HI = 300

WW_E = 327680
WT_E, WD_E, TD_E, TT_E = 81920, 163840, 40960, 20480
GROWS = 53248
CW = 50176
WRANGE = 12544
CT = 3328
CD = 6656
WIN = 64
STG = 1280
WW_SCH = 1024
WW_CB = 1088

_mesh = plsc.VectorSubcoreMesh(core_axis_name="c", subcore_axis_name="s")
_params = pltpu.CompilerParams(needs_layout_passes=False)


def _fill_zero_2d(ref, nrows):
    def body(i, _):
        for c in range(HD // 16):
            ref[i, pl.ds(16 * c, 16)] = jnp.zeros((16,), jnp.float32)
        return 0
    lax.fori_loop(0, nrows, body, 0)


def _fill_zero_1d(ref, n):
    def body(i, _):
        ref[pl.ds(16 * i, 16)] = jnp.zeros((16,), jnp.float32)
        return 0
    lax.fori_loop(0, n // 16, body, 0)


def _fill_ones_1d(ref, n):
    for i in range(n // 16):
        ref[pl.ds(16 * i, 16)] = jnp.ones((16,), jnp.float32)


def _zero_shared_rows(acc_sh, zrows, row0, nrows):
    off = 0
    while off < nrows:
        n = min(WIN, nrows - off)
        pltpu.sync_copy(zrows.at[pl.ds(0, n)], acc_sh.at[pl.ds(row0 + off, n)])
        off += n


def _zero_shared_1d(cnt_sh, zbuf, zlen, start, total):
    off = 0
    while off < total:
        n = min(zlen - (zlen % 8), total - off)
        n = n - (n % 8)
        pltpu.sync_copy(zbuf.at[pl.ds(0, n)], cnt_sh.at[pl.ds(start + off, n)])
        off += n


def _win_engine(h_hbm, acc_sh, cnt_sh, ssrc, sdst, sgain, nwin,
                rows, gsrc, gdst, ones, gsem, ssem, csem,
                counts, static_even_nwin):
    def prep(w, b):
        for g in range(WIN // 16):
            gsrc[b, pl.ds(16 * g, 16)] = ssrc[pl.ds(WIN * w + 16 * g, 16)]
            gdst[b, pl.ds(16 * g, 16)] = sdst[pl.ds(WIN * w + 16 * g, 16)]
        pltpu.async_copy(h_hbm.at[gsrc.at[b]], rows.at[b], gsem.at[b])

    @pl.when(nwin >= 1)
    def _():
        prep(0, 0)

    def pair(t, _):
        for b in (0, 1):
            w = 2 * t + b
            nb = 1 - b

            @pl.when(w < nwin)
            def _():
                pltpu.make_async_copy(h_hbm.at[gsrc.at[b]], rows.at[b],
                                      gsem.at[b]).wait()

                @pl.when(w >= 1)
                def _():
                    pltpu.make_async_copy(rows.at[nb], acc_sh.at[gdst.at[nb]],
                                          ssem.at[nb]).wait()
                    if counts:
                        pltpu.make_async_copy(ones, cnt_sh.at[gdst.at[nb]],
                                              csem.at[nb]).wait()

                @pl.when(w + 1 < nwin)
                def _():
                    prep(w + 1, nb)

                def grp(i, _):
                    g16 = sgain[pl.ds(WIN * w + 16 * i, 16)]
                    for j in range(16):
                        e = 16 * i + j
                        g = g16[j]
                        for c in range(HD // 16):
                            rows[b, e, pl.ds(16 * c, 16)] = (
                                rows[b, e, pl.ds(16 * c, 16)] * g)
                    return 0
                lax.fori_loop(0, WIN // 16, grp, 0)
                pltpu.async_copy(rows.at[b], acc_sh.at[gdst.at[b]],
                                 ssem.at[b], add=True)
                if counts:
                    pltpu.async_copy(ones, cnt_sh.at[gdst.at[b]],
                                     csem.at[b], add=True)
        return 0
    lax.fori_loop(0, (nwin + 1) // 2, pair, 0)

    if static_even_nwin:
        b = (nwin - 1) % 2
        pltpu.make_async_copy(rows.at[b], acc_sh.at[gdst.at[b]],
                              ssem.at[b]).wait()
        if counts:
            pltpu.make_async_copy(ones, cnt_sh.at[gdst.at[b]],
                                  csem.at[b]).wait()
    else:
        for b in (0, 1):
            @pl.when((nwin >= 1) & ((nwin - 1) % 2 == b))
            def _(b=b):
                pltpu.make_async_copy(rows.at[b], acc_sh.at[gdst.at[b]],
                                      ssem.at[b]).wait()


@functools.partial(
    pl.kernel, mesh=_mesh, compiler_params=_params,
    out_type=jax.ShapeDtypeStruct((GROWS, HD), jnp.float32),
    scratch_types=[
        pltpu.VMEM((128,), jnp.int32),
        pltpu.VMEM((128, HD), jnp.float32),
        pltpu.SemaphoreType.DMA,
    ],
)
def _gather_embeds(table_hbm, ids_hbm, out_hbm, idxb, rowsb, sem):
    cid = lax.axis_index("c")
    sid = lax.axis_index("s")
    wid = sid * 2 + cid
    share = GROWS // 32

    def body(w, _):
        base = wid * share + w * 128
        pltpu.sync_copy(ids_hbm.at[pl.ds(base, 128)], idxb)
        pltpu.async_copy(table_hbm.at[idxb], rowsb, sem).wait()
        pltpu.sync_copy(rowsb, out_hbm.at[pl.ds(base, 128)])
        return 0
    lax.fori_loop(0, share // 128, body, 0)


def _make_pair_agg(n_e1, n_e2, cacc, sch1, sch2):
    zlen = cacc // 16

    @functools.partial(
        pl.kernel, mesh=_mesh, compiler_params=_params,
        out_type=(jax.ShapeDtypeStruct((4, cacc, HD), jnp.float32),
                  jax.ShapeDtypeStruct((4 * cacc,), jnp.float32)),
        scratch_types=[
            pltpu.VMEM_SHARED((cacc, HD), jnp.float32),
            pltpu.VMEM_SHARED((cacc, HD), jnp.float32),
            pltpu.VMEM_SHARED((cacc,), jnp.float32),
            pltpu.VMEM_SHARED((cacc,), jnp.float32),
            pltpu.VMEM((STG,), jnp.int32),
            pltpu.VMEM((STG,), jnp.int32),
            pltpu.VMEM((STG,), jnp.float32),
            pltpu.VMEM((WIN,), jnp.float32),
            pltpu.VMEM((2, WIN), jnp.int32),
            pltpu.VMEM((2, WIN), jnp.int32),
            pltpu.VMEM((2, WIN, HD), jnp.float32),
            pltpu.SemaphoreType.DMA((2,)),
            pltpu.SemaphoreType.DMA((2,)),
            pltpu.SemaphoreType.DMA((2,)),
        ],
    )
    def k(s1, d1, w1, ha, s2, d2, w2, hb, out_hbm, outc_hbm,
          accA, accB, cntA, cntB, ssrc, sdst, sgain, ones,
          gsrc, gdst, rows, gsem, ssem, csem):
        cid = lax.axis_index("c")
        sid = lax.axis_index("s")
        wid = sid * 2 + cid
        _fill_ones_1d(ones, WIN)
        _fill_zero_2d(rows.at[0], WIN)
        _fill_zero_1d(sgain, STG)
        chunk = cacc // 16
        _zero_shared_rows(accA, rows.at[0], sid * chunk, chunk)
        _zero_shared_rows(accB, rows.at[0], sid * chunk, chunk)
        _zero_shared_1d(cntA, sgain, STG, sid * zlen, zlen)
        _zero_shared_1d(cntB, sgain, STG, sid * zlen, zlen)
        plsc.subcore_barrier()
        for (src, dst, wgt, h, acc, cnt, n_e, sch) in (
                (s1, d1, w1, ha, accA, cntA, n_e1, sch1),
                (s2, d2, w2, hb, accB, cntB, n_e2, sch2)):
            share = n_e // 32
            nch = share // sch

            def chunk_body(scn, _, src=src, dst=dst, wgt=wgt, h=h,
                           acc=acc, cnt=cnt, share=share, sch=sch):
                base = wid * share + scn * sch
                pltpu.sync_copy(src.at[pl.ds(base, sch)], ssrc.at[pl.ds(0, sch)])
                pltpu.sync_copy(dst.at[pl.ds(base, sch)], sdst.at[pl.ds(0, sch)])
                pltpu.sync_copy(wgt.at[pl.ds(base, sch)], sgain.at[pl.ds(0, sch)])
                _win_engine(h, acc, cnt, ssrc, sdst, sgain, sch // WIN,
                            rows, gsrc, gdst, ones, gsem, ssem, csem,
                            True, True)
                return 0
            lax.fori_loop(0, nch, chunk_body, 0)
        plsc.subcore_barrier()
        pltpu.sync_copy(accA.at[pl.ds(sid * chunk, chunk)],
                        out_hbm.at[cid, pl.ds(sid * chunk, chunk)])
        pltpu.sync_copy(accB.at[pl.ds(sid * chunk, chunk)],
                        out_hbm.at[2 + cid, pl.ds(sid * chunk, chunk)])
        @pl.when(sid == 0)
        def _():
            pltpu.sync_copy(cntA, outc_hbm.at[pl.ds(cid * cacc, cacc)])
            pltpu.sync_copy(cntB, outc_hbm.at[pl.ds((2 + cid) * cacc, cacc)])
    return k


_agg_topic = _make_pair_agg(WT_E, TT_E, CT, STG, 640)
_agg_doc = _make_pair_agg(WD_E, TD_E, CD, STG, STG)


WW_SHARE = WW_E // 16
WW_NCH = WW_SHARE // WW_SCH


@functools.partial(
    pl.kernel, mesh=_mesh, compiler_params=_params,
    out_type=(jax.ShapeDtypeStruct((CW, HD), jnp.float32),
              jax.ShapeDtypeStruct((CW,), jnp.float32)),
    scratch_types=[
        pltpu.VMEM_SHARED((WRANGE, HD), jnp.float32),
        pltpu.VMEM_SHARED((CW,), jnp.float32),
        pltpu.VMEM((WW_SCH,), jnp.int32),
        pltpu.VMEM((WW_SCH,), jnp.int32),
        pltpu.VMEM((WW_SCH,), jnp.float32),
        pltpu.VMEM((WW_CB,), jnp.int32),
        pltpu.VMEM((WW_CB,), jnp.int32),
        pltpu.VMEM((WW_CB,), jnp.float32),
        pltpu.VMEM((WIN,), jnp.float32),
        pltpu.VMEM((2, WIN), jnp.int32),
        pltpu.VMEM((2, WIN), jnp.int32),
        pltpu.VMEM((2, WIN), jnp.int32),
        pltpu.VMEM((2, WIN, HD), jnp.float32),
        pltpu.SemaphoreType.DMA((2,)),
        pltpu.SemaphoreType.DMA((2,)),
        pltpu.SemaphoreType.DMA((2,)),
    ],
)
def _agg_ww(src_hbm, dst_hbm, w_hbm, h_hbm, out_hbm, outc_hbm,
            acc_sh, cnt_sh, srcsc, dstsc, wsc, csrc, cdloc, cgain,
            ones, gsrc, gdst, cidx, rows, gsem, ssem, csem):
    cid = lax.axis_index("c")
    sid = lax.axis_index("s")
    wid = sid * 2 + cid
    _fill_ones_1d(ones, WIN)
    _fill_zero_1d(cgain, WW_CB)
    zc = CW // 16
    _zero_shared_1d(cnt_sh, cgain, WW_CB, sid * zc, zc)
    plsc.subcore_barrier()
    iota16 = lax.broadcasted_iota(jnp.int32, (16,), 0)
    chunk = WRANGE // 16
    for p in range(2):
        lo = (cid * 2 + p) * WRANGE
        _fill_zero_2d(rows.at[0], WIN)
        _zero_shared_rows(acc_sh, rows.at[0], sid * chunk, chunk)
        plsc.subcore_barrier()

        def schunk(scn, _):
            base = sid * WW_SHARE + scn * WW_SCH
            pltpu.sync_copy(src_hbm.at[pl.ds(base, WW_SCH)], srcsc)
            pltpu.sync_copy(dst_hbm.at[pl.ds(base, WW_SCH)], dstsc)
            pltpu.sync_copy(w_hbm.at[pl.ds(base, WW_SCH)], wsc)
            if p == 0:
                def cpair(t, _):
                    for b in (0, 1):
                        w = 2 * t + b

                        @pl.when(jnp.int32(w) >= 2)
                        def _():
                            pltpu.make_async_copy(
                                ones, cnt_sh.at[cidx.at[b]], csem.at[b]).wait()
                        for g in range(WIN // 16):
                            cidx[b, pl.ds(16 * g, 16)] = (
                                dstsc[pl.ds(WIN * w + 16 * g, 16)])
                        pltpu.async_copy(ones, cnt_sh.at[cidx.at[b]],
                                         csem.at[b], add=True)
                    return 0
                lax.fori_loop(0, WW_SCH // WIN // 2, cpair, 0)
                for b in (0, 1):
                    pltpu.make_async_copy(ones, cnt_sh.at[cidx.at[b]],
                                          csem.at[b]).wait()

            def compact(i, pos):
                d16 = dstsc[pl.ds(16 * i, 16)]
                w16 = wsc[pl.ds(16 * i, 16)]
                s16 = srcsc[pl.ds(16 * i, 16)]
                inr = (d16 >= lo) & (d16 < lo + WRANGE)
                plsc.store_compressed(csrc.at[pl.ds(pos, 16)], s16, mask=inr)
                plsc.store_compressed(cdloc.at[pl.ds(pos, 16)], d16 - lo, mask=inr)
                plsc.store_compressed(cgain.at[pl.ds(pos, 16)], w16, mask=inr)
                return pos + plsc.all_reduce_population_count(inr)[0]
            pos = lax.fori_loop(0, WW_SCH // 16, compact, jnp.int32(0))
            for kk in range(WIN // 16):
                pad_idx = iota16 + (16 * kk + wid * 128)
                csrc[pl.ds(pos + 16 * kk, 16)] = pad_idx
                cdloc[pl.ds(pos + 16 * kk, 16)] = jnp.zeros((16,), jnp.int32)
                cgain[pl.ds(pos + 16 * kk, 16)] = jnp.zeros((16,), jnp.float32)
            _win_engine(h_hbm, acc_sh, None, csrc, cdloc, cgain,
                        (pos + WIN - 1) // WIN,
                        rows, gsrc, gdst, ones, gsem, ssem, csem,
                        False, False)
            return 0
        lax.fori_loop(0, WW_NCH, schunk, 0)
        plsc.subcore_barrier()
        pltpu.sync_copy(acc_sh.at[pl.ds(sid * chunk, chunk)],
                        out_hbm.at[pl.ds(lo + sid * chunk, chunk)])
        if p == 0:
            @pl.when((cid == 0) & (sid == 0))
            def _():
                pltpu.sync_copy(cnt_sh, outc_hbm)
        plsc.subcore_barrier()


def _prep_body(aW, ab, cW, cb, tids, temb,
               m0_o, b0_o, m1_o, b1_o, t1_o, tb1_o, ht0_o):
    m0 = aW[...]
    b0 = ab[...]
    for i in range(3):
        w = cW[0, i]
        m0 = jnp.dot(m0, w, preferred_element_type=jnp.float32)
        b0 = jnp.dot(b0, w, preferred_element_type=jnp.float32) + cb[0, i]
    m1 = cW[1, 0]
    b1 = cb[1, 0]
    for i in range(1, 3):
        w = cW[1, i]
        m1 = jnp.dot(m1, w, preferred_element_type=jnp.float32)
        b1 = jnp.dot(b1, w, preferred_element_type=jnp.float32) + cb[1, i]
    t1 = jnp.dot(cW[1, 3], cW[1, 4], preferred_element_type=jnp.float32)
    tb1 = jnp.dot(cb[1, 3], cW[1, 4], preferred_element_type=jnp.float32) + cb[1, 4]
    tt0 = jnp.dot(temb[...], cW[0, 3], preferred_element_type=jnp.float32) + cb[0, 3]
    tt0 = jnp.dot(tt0, cW[0, 4], preferred_element_type=jnp.float32) + cb[0, 4]
    ids = tids[...]
    oh = (lax.broadcasted_iota(jnp.int32, (NTN, 64), 1) == ids).astype(jnp.float32)
    ht0_o[...] = jnp.dot(oh, tt0, preferred_element_type=jnp.float32)
    m0_o[...] = m0
    b0_o[...] = b0
    m1_o[...] = m1
    b1_o[...] = b1
    t1_o[...] = t1
    tb1_o[...] = tb1


def _prep(aW, ab, cW, cb, tids, temb):
    return pl.pallas_call(
        _prep_body,
        out_shape=(
            jax.ShapeDtypeStruct((HI, HD), jnp.float32),
            jax.ShapeDtypeStruct((1, HD), jnp.float32),
            jax.ShapeDtypeStruct((HD, HD), jnp.float32),
            jax.ShapeDtypeStruct((1, HD), jnp.float32),
            jax.ShapeDtypeStruct((HD, HD), jnp.float32),
            jax.ShapeDtypeStruct((1, HD), jnp.float32),
            jax.ShapeDtypeStruct((NTN, HD), jnp.float32),
        ),
    )(aW, ab, cW, cb, tids, temb)


def _matmul_bias(x, m, b, tr=512):
    n, k = x.shape
    assert n % tr == 0

    def body(x_ref, m_ref, b_ref, o_ref):
        o_ref[...] = jnp.dot(x_ref[...], m_ref[...],
                             preferred_element_type=jnp.float32) + b_ref[...]

    return pl.pallas_call(
        body,
        grid=(n // tr,),
        in_specs=[
            pl.BlockSpec((tr, k), lambda i: (i, 0)),
            pl.BlockSpec((k, HD), lambda i: (0, 0)),
            pl.BlockSpec((1, HD), lambda i: (0, 0)),
        ],
        out_specs=pl.BlockSpec((tr, HD), lambda i: (i, 0)),
        out_shape=jax.ShapeDtypeStruct((n, HD), jnp.float32),
    )(x, m, b)


def _scale_relu_matmul(x, c, m, b, tr=512):
    n, k = x.shape
    assert n % tr == 0

    def body(x_ref, c_ref, m_ref, b_ref, o_ref):
        r = 1.0 / jnp.maximum(c_ref[...], 1.0)
        h = jnp.maximum(x_ref[...], 0.0) * r
        o_ref[...] = jnp.dot(h, m_ref[...],
                             preferred_element_type=jnp.float32) + b_ref[...]

    return pl.pallas_call(
        body,
        grid=(n // tr,),
        in_specs=[
            pl.BlockSpec((tr, k), lambda i: (i, 0)),
            pl.BlockSpec((tr, 1), lambda i: (i, 0)),
            pl.BlockSpec((k, HD), lambda i: (0, 0)),
            pl.BlockSpec((1, HD), lambda i: (0, 0)),
        ],
        out_specs=pl.BlockSpec((tr, HD), lambda i: (i, 0)),
        out_shape=jax.ShapeDtypeStruct((n, HD), jnp.float32),
    )(x, c, m, b)


def _combine_pair_matmul(acc, cnt, m, b):
    _, n, _ = acc.shape

    def body(a_ref, c_ref, m_ref, b_ref, o_ref):
        ra = 1.0 / jnp.maximum(c_ref[0] + c_ref[1], 1.0)
        rb = 1.0 / jnp.maximum(c_ref[2] + c_ref[3], 1.0)
        h = jnp.maximum((a_ref[0] + a_ref[1]) * ra
                        + (a_ref[2] + a_ref[3]) * rb, 0.0)
        o_ref[...] = jnp.dot(h, m_ref[...],
                             preferred_element_type=jnp.float32) + b_ref[...]

    return pl.pallas_call(
        body,
        out_shape=jax.ShapeDtypeStruct((n, HD), jnp.float32),
    )(acc, cnt, m, b)


def _final(acc, cnt, ow, ob, y):
    def body(a_ref, c_ref, ow_ref, ob_ref, y_ref, loss_ref, yp_ref):
        ra = 1.0 / jnp.maximum(c_ref[0] + c_ref[1], 1.0)
        rb = 1.0 / jnp.maximum(c_ref[2] + c_ref[3], 1.0)
        hd = jnp.maximum((a_ref[0] + a_ref[1]) * ra
                         + (a_ref[2] + a_ref[3]) * rb, 0.0)
        ms = []
        for g in range(NB):
            ms.append(jnp.max(hd[100 * g:100 * g + 100, :], axis=0,
                              keepdims=True))
        glob = jnp.concatenate(ms, axis=0)
        z = jnp.sum(glob * ow_ref[...], axis=1, keepdims=True) + ob_ref[...]
        yv = y_ref[...]
        lossv = jnp.mean(jnp.maximum(z, 0.0) - z * yv
                         + jnp.log(1.0 + jnp.exp(-jnp.abs(z))))
        loss_ref[...] = lossv[None, None]
        yp_ref[...] = 1.0 / (1.0 + jnp.exp(-z))

    return pl.pallas_call(
        body,
        out_shape=(
            jax.ShapeDtypeStruct((1, 1), jnp.float32),
            jax.ShapeDtypeStruct((NB, 1), jnp.float32),
        ),
    )(acc, cnt, ow, ob, y)


def _pad_edges(src, dst, w, n_pad, n_src, n_dst):
    e = src.shape[0]
    k = n_pad - e
    pad_src = (jnp.arange(k, dtype=jnp.int32) * 7919) % n_src
    src = jnp.concatenate([src.astype(jnp.int32), pad_src])
    dst = jnp.concatenate([dst.astype(jnp.int32),
                           jnp.full((k,), n_dst, jnp.int32)])
    w = jnp.concatenate([w, jnp.zeros((k,), w.dtype)])
    return src, dst, w


def kernel(word_ids, topic_ids, ww_src, ww_dst, ww_w, wt_src, wt_dst, wt_w,
           wd_src, wd_dst, wd_w, td_src, td_dst, td_w, tt_src, tt_dst, tt_w,
           doc_graph_ids, y_data, word_embeds, topic_embeds, adapt_W, adapt_b,
           conv_W, conv_b, out_W, out_b):
    wid_pad = jnp.concatenate([
        word_ids.astype(jnp.int32),
        (jnp.arange(GROWS - NWN, dtype=jnp.int32) * 7919) % VOC])
    ww = _pad_edges(ww_src, ww_dst, ww_w, WW_E, NWN, NWN)
    wt = _pad_edges(wt_src, wt_dst, wt_w, WT_E, NWN, NTN)
    wd = _pad_edges(wd_src, wd_dst, wd_w, WD_E, NWN, NDN)
    td = _pad_edges(td_src, td_dst, td_w, TD_E, NTN, NDN)
    tt = _pad_edges(tt_src, tt_dst, tt_w, TT_E, NTN, NTN)
    temb_pad = jnp.concatenate(
        [topic_embeds, jnp.zeros((14, HD), jnp.float32)], axis=0)
    cb4 = conv_b.reshape(2, 5, 1, HD)
    ab2 = adapt_b.reshape(1, HD)
    tids2 = topic_ids.astype(jnp.int32).reshape(NTN, 1)
    y2 = y_data.reshape(NB, 1)
    ow2 = out_W.reshape(1, HD)
    ob2 = out_b.reshape(1, 1)

    m0, b0, m1, b1, t1, tb1, ht0 = _prep(adapt_W, ab2, conv_W, cb4, tids2, temb_pad)

    tword = _matmul_bias(word_embeds, m0, b0, tr=1000)
    hw0 = _gather_embeds(tword, wid_pad)

    at, atc = _agg_topic(wt[0], wt[1], wt[2], hw0, tt[0], tt[1], tt[2], ht0)
    aww, awwc = _agg_ww(ww[0], ww[1], ww[2], hw0)

    hw1 = _scale_relu_matmul(aww, awwc.reshape(CW, 1), m1, b1)
    ht1 = _combine_pair_matmul(at, atc.reshape(4, CT, 1), t1, tb1)

    ad, adc = _agg_doc(wd[0], wd[1], wd[2], hw1, td[0], td[1], td[2], ht1)

    loss2, yp = _final(ad, adc.reshape(4, CD, 1), ow2, ob2, y2)
    return (loss2.reshape(()), yp)


# speedup vs baseline: 3.7980x; 1.0290x over previous
import functools

import jax
import jax.numpy as jnp
from jax import lax
from jax.experimental import pallas as pl
from jax.experimental.pallas import tpu as pltpu
from jax.experimental.pallas import tpu_sc as plsc

HD = 128
NWN = 50000
NTN = 3200
NDN = 6400
NB = 64
VOC = 100000
HI = 300

WW_E = 327680
WT_E, WD_E, TD_E, TT_E = 81920, 163840, 40960, 20480
GROWS = 53248
CW = 50176
WRANGE = 12544
CT = 3328
CD = 6656
WIN = 64
STG = 1280
WW_SCH = 1024
WW_CB = 1088

_mesh = plsc.VectorSubcoreMesh(core_axis_name="c", subcore_axis_name="s")
_params = pltpu.CompilerParams(needs_layout_passes=False)


def _fill_zero_2d(ref, nrows):
    def body(i, _):
        for c in range(HD // 16):
            ref[i, pl.ds(16 * c, 16)] = jnp.zeros((16,), jnp.float32)
        return 0
    lax.fori_loop(0, nrows, body, 0)


def _fill_zero_1d(ref, n):
    def body(i, _):
        ref[pl.ds(16 * i, 16)] = jnp.zeros((16,), jnp.float32)
        return 0
    lax.fori_loop(0, n // 16, body, 0)


def _fill_ones_1d(ref, n):
    for i in range(n // 16):
        ref[pl.ds(16 * i, 16)] = jnp.ones((16,), jnp.float32)


def _zero_shared_rows(acc_sh, zrows, row0, nrows, win=WIN):
    off = 0
    while off < nrows:
        n = min(win, nrows - off)
        pltpu.sync_copy(zrows.at[pl.ds(0, n)], acc_sh.at[pl.ds(row0 + off, n)])
        off += n


def _zero_shared_1d(cnt_sh, zbuf, zlen, start, total):
    off = 0
    while off < total:
        n = min(zlen - (zlen % 8), total - off)
        n = n - (n % 8)
        pltpu.sync_copy(zbuf.at[pl.ds(0, n)], cnt_sh.at[pl.ds(start + off, n)])
        off += n


def _win_engine(h_hbm, acc_sh, cnt_sh, ssrc, sdst, sgain, nwin,
                rows, gsrc, gdst, ones, gsem, ssem, csem,
                counts, static_nwin, win=WIN):
    WIN = win

    def prep(w, b):
        for g in range(WIN // 16):
            gsrc[b, pl.ds(16 * g, 16)] = ssrc[pl.ds(WIN * w + 16 * g, 16)]
            gdst[b, pl.ds(16 * g, 16)] = sdst[pl.ds(WIN * w + 16 * g, 16)]
        pltpu.async_copy(h_hbm.at[gsrc.at[b]], rows.at[b], gsem.at[b])

    @pl.when(nwin >= 1)
    def _():
        prep(0, 0)

    def pair(t, _):
        for b in (0, 1):
            w = 2 * t + b
            nb = 1 - b

            @pl.when(w < nwin)
            def _():
                pltpu.make_async_copy(h_hbm.at[gsrc.at[b]], rows.at[b],
                                      gsem.at[b]).wait()

                @pl.when(w >= 1)
                def _():
                    pltpu.make_async_copy(rows.at[nb], acc_sh.at[gdst.at[nb]],
                                          ssem.at[nb]).wait()
                    if counts:
                        pltpu.make_async_copy(ones, cnt_sh.at[gdst.at[nb]],
                                              csem.at[nb]).wait()

                @pl.when(w + 1 < nwin)
                def _():
                    prep(w + 1, nb)

                def grp(i, _):
                    g16 = sgain[pl.ds(WIN * w + 16 * i, 16)]
                    for j in range(16):
                        e = 16 * i + j
                        g = g16[j]
                        for c in range(HD // 16):
                            rows[b, e, pl.ds(16 * c, 16)] = (
                                rows[b, e, pl.ds(16 * c, 16)] * g)
                    return 0
                lax.fori_loop(0, WIN // 16, grp, 0)
                pltpu.async_copy(rows.at[b], acc_sh.at[gdst.at[b]],
                                 ssem.at[b], add=True)
                if counts:
                    pltpu.async_copy(ones, cnt_sh.at[gdst.at[b]],
                                     csem.at[b], add=True)
        return 0
    lax.fori_loop(0, (nwin + 1) // 2, pair, 0)

    if static_nwin:
        b = (nwin - 1) % 2
        pltpu.make_async_copy(rows.at[b], acc_sh.at[gdst.at[b]],
                              ssem.at[b]).wait()
        if counts:
            pltpu.make_async_copy(ones, cnt_sh.at[gdst.at[b]],
                                  csem.at[b]).wait()
    else:
        for b in (0, 1):
            @pl.when((nwin >= 1) & ((nwin - 1) % 2 == b))
            def _(b=b):
                pltpu.make_async_copy(rows.at[b], acc_sh.at[gdst.at[b]],
                                      ssem.at[b]).wait()


@functools.partial(
    pl.kernel, mesh=_mesh, compiler_params=_params,
    out_type=jax.ShapeDtypeStruct((GROWS, HD), jnp.float32),
    scratch_types=[
        pltpu.VMEM((128,), jnp.int32),
        pltpu.VMEM((128, HD), jnp.float32),
        pltpu.SemaphoreType.DMA,
    ],
)
def _gather_embeds(table_hbm, ids_hbm, out_hbm, idxb, rowsb, sem):
    cid = lax.axis_index("c")
    sid = lax.axis_index("s")
    wid = sid * 2 + cid
    share = GROWS // 32

    def body(w, _):
        base = wid * share + w * 128
        pltpu.sync_copy(ids_hbm.at[pl.ds(base, 128)], idxb)
        pltpu.async_copy(table_hbm.at[idxb], rowsb, sem).wait()
        pltpu.sync_copy(rowsb, out_hbm.at[pl.ds(base, 128)])
        return 0
    lax.fori_loop(0, share // 128, body, 0)


def _make_pair_agg(n_e1, n_e2, cacc, sch1, sch2):
    zlen = cacc // 16
    PW = 128
    stg = max(sch1, sch2)

    @functools.partial(
        pl.kernel, mesh=_mesh, compiler_params=_params,
        out_type=(jax.ShapeDtypeStruct((4, cacc, HD), jnp.float32),
                  jax.ShapeDtypeStruct((4 * cacc,), jnp.float32)),
        scratch_types=[
            pltpu.VMEM_SHARED((cacc, HD), jnp.float32),
            pltpu.VMEM_SHARED((cacc,), jnp.float32),
            pltpu.VMEM_SHARED((cacc,), jnp.float32),
            pltpu.VMEM((stg,), jnp.int32),
            pltpu.VMEM((stg,), jnp.int32),
            pltpu.VMEM((stg,), jnp.float32),
            pltpu.VMEM((PW,), jnp.float32),
            pltpu.VMEM((2, PW), jnp.int32),
            pltpu.VMEM((2, PW), jnp.int32),
            pltpu.VMEM((2, PW, HD), jnp.float32),
            pltpu.SemaphoreType.DMA((2,)),
            pltpu.SemaphoreType.DMA((2,)),
            pltpu.SemaphoreType.DMA((2,)),
        ],
    )
    def k(s1, d1, w1, ha, s2, d2, w2, hb, out_hbm, outc_hbm,
          acc, cntA, cntB, ssrc, sdst, sgain, ones,
          gsrc, gdst, rows, gsem, ssem, csem):
        cid = lax.axis_index("c")
        sid = lax.axis_index("s")
        wid = sid * 2 + cid
        _fill_ones_1d(ones, PW)
        _fill_zero_1d(sgain, stg)
        chunk = cacc // 16
        _zero_shared_1d(cntA, sgain, stg, sid * zlen, zlen)
        _zero_shared_1d(cntB, sgain, stg, sid * zlen, zlen)
        for ei, (src, dst, wgt, h, cnt, n_e, sch) in enumerate((
                (s1, d1, w1, ha, cntA, n_e1, sch1),
                (s2, d2, w2, hb, cntB, n_e2, sch2))):
            _fill_zero_2d(rows.at[0], PW)
            _zero_shared_rows(acc, rows.at[0], sid * chunk, chunk, PW)
            plsc.subcore_barrier()
            share = n_e // 32
            nch = share // sch

            def chunk_body(scn, _, src=src, dst=dst, wgt=wgt, h=h,
                           cnt=cnt, share=share, sch=sch):
                base = wid * share + scn * sch
                pltpu.sync_copy(src.at[pl.ds(base, sch)], ssrc.at[pl.ds(0, sch)])
                pltpu.sync_copy(dst.at[pl.ds(base, sch)], sdst.at[pl.ds(0, sch)])
                pltpu.sync_copy(wgt.at[pl.ds(base, sch)], sgain.at[pl.ds(0, sch)])
                _win_engine(h, acc, cnt, ssrc, sdst, sgain, sch // PW,
                            rows, gsrc, gdst, ones, gsem, ssem, csem,
                            True, True, win=PW)
                return 0
            lax.fori_loop(0, nch, chunk_body, 0)
            plsc.subcore_barrier()
            pltpu.sync_copy(acc.at[pl.ds(sid * chunk, chunk)],
                            out_hbm.at[2 * ei + cid, pl.ds(sid * chunk, chunk)])
            plsc.subcore_barrier()
        @pl.when(sid == 0)
        def _():
            pltpu.sync_copy(cntA, outc_hbm.at[pl.ds(cid * cacc, cacc)])
            pltpu.sync_copy(cntB, outc_hbm.at[pl.ds((2 + cid) * cacc, cacc)])
    return k


_agg_topic = _make_pair_agg(WT_E, TT_E, CT, 2560, 640)
_agg_doc = _make_pair_agg(WD_E, TD_E, CD, 2560, 1280)


WW_SHARE = WW_E // 16
WW_NCH = WW_SHARE // WW_SCH


@functools.partial(
    pl.kernel, mesh=_mesh, compiler_params=_params,
    out_type=(jax.ShapeDtypeStruct((CW, HD), jnp.float32),
              jax.ShapeDtypeStruct((CW,), jnp.float32)),
    scratch_types=[
        pltpu.VMEM_SHARED((WRANGE, HD), jnp.float32),
        pltpu.VMEM_SHARED((CW,), jnp.float32),
        pltpu.VMEM((WW_SCH,), jnp.int32),
        pltpu.VMEM((WW_SCH,), jnp.int32),
        pltpu.VMEM((WW_SCH,), jnp.float32),
        pltpu.VMEM((WW_CB,), jnp.int32),
        pltpu.VMEM((WW_CB,), jnp.int32),
        pltpu.VMEM((WW_CB,), jnp.float32),
        pltpu.VMEM((WIN,), jnp.float32),
        pltpu.VMEM((2, WIN), jnp.int32),
        pltpu.VMEM((2, WIN), jnp.int32),
        pltpu.VMEM((2, WIN), jnp.int32),
        pltpu.VMEM((2, WIN, HD), jnp.float32),
        pltpu.SemaphoreType.DMA((2,)),
        pltpu.SemaphoreType.DMA((2,)),
        pltpu.SemaphoreType.DMA((2,)),
    ],
)
def _agg_ww(src_hbm, dst_hbm, w_hbm, h_hbm, out_hbm, outc_hbm,
            acc_sh, cnt_sh, srcsc, dstsc, wsc, csrc, cdloc, cgain,
            ones, gsrc, gdst, cidx, rows, gsem, ssem, csem):
    cid = lax.axis_index("c")
    sid = lax.axis_index("s")
    wid = sid * 2 + cid
    _fill_ones_1d(ones, WIN)
    _fill_zero_1d(cgain, WW_CB)
    zc = CW // 16
    _zero_shared_1d(cnt_sh, cgain, WW_CB, sid * zc, zc)
    plsc.subcore_barrier()
    iota16 = lax.broadcasted_iota(jnp.int32, (16,), 0)
    chunk = WRANGE // 16
    for p in range(2):
        lo = (cid * 2 + p) * WRANGE
        _fill_zero_2d(rows.at[0], WIN)
        _zero_shared_rows(acc_sh, rows.at[0], sid * chunk, chunk)
        plsc.subcore_barrier()

        def schunk(scn, _):
            base = sid * WW_SHARE + scn * WW_SCH
            pltpu.sync_copy(src_hbm.at[pl.ds(base, WW_SCH)], srcsc)
            pltpu.sync_copy(dst_hbm.at[pl.ds(base, WW_SCH)], dstsc)
            pltpu.sync_copy(w_hbm.at[pl.ds(base, WW_SCH)], wsc)
            if p == 0:
                def cpair(t, _):
                    for b in (0, 1):
                        w = 2 * t + b

                        @pl.when(jnp.int32(w) >= 2)
                        def _():
                            pltpu.make_async_copy(
                                ones, cnt_sh.at[cidx.at[b]], csem.at[b]).wait()
                        for g in range(WIN // 16):
                            cidx[b, pl.ds(16 * g, 16)] = (
                                dstsc[pl.ds(WIN * w + 16 * g, 16)])
                        pltpu.async_copy(ones, cnt_sh.at[cidx.at[b]],
                                         csem.at[b], add=True)
                    return 0
                lax.fori_loop(0, WW_SCH // WIN // 2, cpair, 0)
                for b in (0, 1):
                    pltpu.make_async_copy(ones, cnt_sh.at[cidx.at[b]],
                                          csem.at[b]).wait()

            def compact(i, pos):
                d16 = dstsc[pl.ds(16 * i, 16)]
                w16 = wsc[pl.ds(16 * i, 16)]
                s16 = srcsc[pl.ds(16 * i, 16)]
                inr = (d16 >= lo) & (d16 < lo + WRANGE)
                plsc.store_compressed(csrc.at[pl.ds(pos, 16)], s16, mask=inr)
                plsc.store_compressed(cdloc.at[pl.ds(pos, 16)], d16 - lo, mask=inr)
                plsc.store_compressed(cgain.at[pl.ds(pos, 16)], w16, mask=inr)
                return pos + plsc.all_reduce_population_count(inr)[0]
            pos = lax.fori_loop(0, WW_SCH // 16, compact, jnp.int32(0))
            for kk in range(WIN // 16):
                pad_idx = iota16 + (16 * kk + wid * 128)
                csrc[pl.ds(pos + 16 * kk, 16)] = pad_idx
                cdloc[pl.ds(pos + 16 * kk, 16)] = jnp.zeros((16,), jnp.int32)
                cgain[pl.ds(pos + 16 * kk, 16)] = jnp.zeros((16,), jnp.float32)
            _win_engine(h_hbm, acc_sh, None, csrc, cdloc, cgain,
                        (pos + WIN - 1) // WIN,
                        rows, gsrc, gdst, ones, gsem, ssem, csem,
                        False, False)
            return 0
        lax.fori_loop(0, WW_NCH, schunk, 0)
        plsc.subcore_barrier()
        pltpu.sync_copy(acc_sh.at[pl.ds(sid * chunk, chunk)],
                        out_hbm.at[pl.ds(lo + sid * chunk, chunk)])
        if p == 0:
            @pl.when((cid == 0) & (sid == 0))
            def _():
                pltpu.sync_copy(cnt_sh, outc_hbm)
        plsc.subcore_barrier()


def _prep_body(aW, ab, cW, cb, tids, temb,
               m0_o, b0_o, m1_o, b1_o, t1_o, tb1_o, ht0_o):
    m0 = aW[...]
    b0 = ab[...]
    for i in range(3):
        w = cW[0, i]
        m0 = jnp.dot(m0, w, preferred_element_type=jnp.float32)
        b0 = jnp.dot(b0, w, preferred_element_type=jnp.float32) + cb[0, i]
    m1 = cW[1, 0]
    b1 = cb[1, 0]
    for i in range(1, 3):
        w = cW[1, i]
        m1 = jnp.dot(m1, w, preferred_element_type=jnp.float32)
        b1 = jnp.dot(b1, w, preferred_element_type=jnp.float32) + cb[1, i]
    t1 = jnp.dot(cW[1, 3], cW[1, 4], preferred_element_type=jnp.float32)
    tb1 = jnp.dot(cb[1, 3], cW[1, 4], preferred_element_type=jnp.float32) + cb[1, 4]
    tt0 = jnp.dot(temb[...], cW[0, 3], preferred_element_type=jnp.float32) + cb[0, 3]
    tt0 = jnp.dot(tt0, cW[0, 4], preferred_element_type=jnp.float32) + cb[0, 4]
    ids = tids[...]
    oh = (lax.broadcasted_iota(jnp.int32, (NTN, 64), 1) == ids).astype(jnp.float32)
    ht0_o[...] = jnp.dot(oh, tt0, preferred_element_type=jnp.float32)
    m0_o[...] = m0
    b0_o[...] = b0
    m1_o[...] = m1
    b1_o[...] = b1
    t1_o[...] = t1
    tb1_o[...] = tb1


def _prep(aW, ab, cW, cb, tids, temb):
    return pl.pallas_call(
        _prep_body,
        out_shape=(
            jax.ShapeDtypeStruct((HI, HD), jnp.float32),
            jax.ShapeDtypeStruct((1, HD), jnp.float32),
            jax.ShapeDtypeStruct((HD, HD), jnp.float32),
            jax.ShapeDtypeStruct((1, HD), jnp.float32),
            jax.ShapeDtypeStruct((HD, HD), jnp.float32),
            jax.ShapeDtypeStruct((1, HD), jnp.float32),
            jax.ShapeDtypeStruct((NTN, HD), jnp.float32),
        ),
    )(aW, ab, cW, cb, tids, temb)


def _matmul_bias(x, m, b, tr=512):
    n, k = x.shape
    assert n % tr == 0

    def body(x_ref, m_ref, b_ref, o_ref):
        o_ref[...] = jnp.dot(x_ref[...], m_ref[...],
                             preferred_element_type=jnp.float32) + b_ref[...]

    return pl.pallas_call(
        body,
        grid=(n // tr,),
        in_specs=[
            pl.BlockSpec((tr, k), lambda i: (i, 0)),
            pl.BlockSpec((k, HD), lambda i: (0, 0)),
            pl.BlockSpec((1, HD), lambda i: (0, 0)),
        ],
        out_specs=pl.BlockSpec((tr, HD), lambda i: (i, 0)),
        out_shape=jax.ShapeDtypeStruct((n, HD), jnp.float32),
    )(x, m, b)


def _scale_relu_matmul(x, c, m, b, tr=512):
    n, k = x.shape
    assert n % tr == 0

    def body(x_ref, c_ref, m_ref, b_ref, o_ref):
        r = 1.0 / jnp.maximum(c_ref[...], 1.0)
        h = jnp.maximum(x_ref[...], 0.0) * r
        o_ref[...] = jnp.dot(h, m_ref[...],
                             preferred_element_type=jnp.float32) + b_ref[...]

    return pl.pallas_call(
        body,
        grid=(n // tr,),
        in_specs=[
            pl.BlockSpec((tr, k), lambda i: (i, 0)),
            pl.BlockSpec((tr, 1), lambda i: (i, 0)),
            pl.BlockSpec((k, HD), lambda i: (0, 0)),
            pl.BlockSpec((1, HD), lambda i: (0, 0)),
        ],
        out_specs=pl.BlockSpec((tr, HD), lambda i: (i, 0)),
        out_shape=jax.ShapeDtypeStruct((n, HD), jnp.float32),
    )(x, c, m, b)


def _combine_pair_matmul(acc, cnt, m, b):
    _, n, _ = acc.shape

    def body(a_ref, c_ref, m_ref, b_ref, o_ref):
        ra = 1.0 / jnp.maximum(c_ref[0] + c_ref[1], 1.0)
        rb = 1.0 / jnp.maximum(c_ref[2] + c_ref[3], 1.0)
        h = jnp.maximum((a_ref[0] + a_ref[1]) * ra
                        + (a_ref[2] + a_ref[3]) * rb, 0.0)
        o_ref[...] = jnp.dot(h, m_ref[...],
                             preferred_element_type=jnp.float32) + b_ref[...]

    return pl.pallas_call(
        body,
        out_shape=jax.ShapeDtypeStruct((n, HD), jnp.float32),
    )(acc, cnt, m, b)


def _final(acc, cnt, ow, ob, y):
    def body(a_ref, c_ref, ow_ref, ob_ref, y_ref, loss_ref, yp_ref):
        ra = 1.0 / jnp.maximum(c_ref[0] + c_ref[1], 1.0)
        rb = 1.0 / jnp.maximum(c_ref[2] + c_ref[3], 1.0)
        hd = jnp.maximum((a_ref[0] + a_ref[1]) * ra
                         + (a_ref[2] + a_ref[3]) * rb, 0.0)
        ms = []
        for g in range(NB):
            ms.append(jnp.max(hd[100 * g:100 * g + 100, :], axis=0,
                              keepdims=True))
        glob = jnp.concatenate(ms, axis=0)
        z = jnp.sum(glob * ow_ref[...], axis=1, keepdims=True) + ob_ref[...]
        yv = y_ref[...]
        lossv = jnp.mean(jnp.maximum(z, 0.0) - z * yv
                         + jnp.log(1.0 + jnp.exp(-jnp.abs(z))))
        loss_ref[...] = lossv[None, None]
        yp_ref[...] = 1.0 / (1.0 + jnp.exp(-z))

    return pl.pallas_call(
        body,
        out_shape=(
            jax.ShapeDtypeStruct((1, 1), jnp.float32),
            jax.ShapeDtypeStruct((NB, 1), jnp.float32),
        ),
    )(acc, cnt, ow, ob, y)


def _pad_edges(src, dst, w, n_pad, n_src, n_dst):
    e = src.shape[0]
    k = n_pad - e
    pad_src = (jnp.arange(k, dtype=jnp.int32) * 7919) % n_src
    src = jnp.concatenate([src.astype(jnp.int32), pad_src])
    dst = jnp.concatenate([dst.astype(jnp.int32),
                           jnp.full((k,), n_dst, jnp.int32)])
    w = jnp.concatenate([w, jnp.zeros((k,), w.dtype)])
    return src, dst, w


def kernel(word_ids, topic_ids, ww_src, ww_dst, ww_w, wt_src, wt_dst, wt_w,
           wd_src, wd_dst, wd_w, td_src, td_dst, td_w, tt_src, tt_dst, tt_w,
           doc_graph_ids, y_data, word_embeds, topic_embeds, adapt_W, adapt_b,
           conv_W, conv_b, out_W, out_b):
    wid_pad = jnp.concatenate([
        word_ids.astype(jnp.int32),
        (jnp.arange(GROWS - NWN, dtype=jnp.int32) * 7919) % VOC])
    ww = _pad_edges(ww_src, ww_dst, ww_w, WW_E, NWN, NWN)
    wt = _pad_edges(wt_src, wt_dst, wt_w, WT_E, NWN, NTN)
    wd = _pad_edges(wd_src, wd_dst, wd_w, WD_E, NWN, NDN)
    td = _pad_edges(td_src, td_dst, td_w, TD_E, NTN, NDN)
    tt = _pad_edges(tt_src, tt_dst, tt_w, TT_E, NTN, NTN)
    temb_pad = jnp.concatenate(
        [topic_embeds, jnp.zeros((14, HD), jnp.float32)], axis=0)
    cb4 = conv_b.reshape(2, 5, 1, HD)
    ab2 = adapt_b.reshape(1, HD)
    tids2 = topic_ids.astype(jnp.int32).reshape(NTN, 1)
    y2 = y_data.reshape(NB, 1)
    ow2 = out_W.reshape(1, HD)
    ob2 = out_b.reshape(1, 1)

    m0, b0, m1, b1, t1, tb1, ht0 = _prep(adapt_W, ab2, conv_W, cb4, tids2, temb_pad)

    tword = _matmul_bias(word_embeds, m0, b0, tr=1000)
    hw0 = _gather_embeds(tword, wid_pad)

    at, atc = _agg_topic(wt[0], wt[1], wt[2], hw0, tt[0], tt[1], tt[2], ht0)
    aww, awwc = _agg_ww(ww[0], ww[1], ww[2], hw0)

    hw1 = _scale_relu_matmul(aww, awwc.reshape(CW, 1), m1, b1)
    ht1 = _combine_pair_matmul(at, atc.reshape(4, CT, 1), t1, tb1)

    ad, adc = _agg_doc(wd[0], wd[1], wd[2], hw1, td[0], td[1], td[2], ht1)

    loss2, yp = _final(ad, adc.reshape(4, CD, 1), ow2, ob2, y2)
    return (loss2.reshape(()), yp)


# speedup vs baseline: 3.9319x; 1.0353x over previous
import functools

import jax
import jax.numpy as jnp
from jax import lax
from jax.experimental import pallas as pl
from jax.experimental.pallas import tpu as pltpu
from jax.experimental.pallas import tpu_sc as plsc

HD = 128
NWN = 50000
NTN = 3200
NDN = 6400
NB = 64
VOC = 100000
HI = 300

WW_E = 327680
WT_E, WD_E, TD_E, TT_E = 81920, 163840, 40960, 20480
GROWS = 53248
CW = 50176
WRANGE = 12544
CT = 3328
CD = 6656
WIN = 64
STG = 1280
WW_SCH = 1280
WW_CB = 1344

_mesh = plsc.VectorSubcoreMesh(core_axis_name="c", subcore_axis_name="s")
_params = pltpu.CompilerParams(needs_layout_passes=False)


def _fill_zero_2d(ref, nrows):
    def body(i, _):
        for c in range(HD // 16):
            ref[i, pl.ds(16 * c, 16)] = jnp.zeros((16,), jnp.float32)
        return 0
    lax.fori_loop(0, nrows, body, 0)


def _fill_zero_1d(ref, n):
    def body(i, _):
        ref[pl.ds(16 * i, 16)] = jnp.zeros((16,), jnp.float32)
        return 0
    lax.fori_loop(0, n // 16, body, 0)


def _fill_ones_1d(ref, n):
    for i in range(n // 16):
        ref[pl.ds(16 * i, 16)] = jnp.ones((16,), jnp.float32)


def _zero_shared_rows(acc_sh, zrows, row0, nrows, win=WIN):
    off = 0
    while off < nrows:
        n = min(win, nrows - off)
        pltpu.sync_copy(zrows.at[pl.ds(0, n)], acc_sh.at[pl.ds(row0 + off, n)])
        off += n


def _zero_shared_1d(cnt_sh, zbuf, zlen, start, total):
    off = 0
    while off < total:
        n = min(zlen - (zlen % 8), total - off)
        n = n - (n % 8)
        pltpu.sync_copy(zbuf.at[pl.ds(0, n)], cnt_sh.at[pl.ds(start + off, n)])
        off += n


def _win_engine(h_hbm, acc_sh, cnt_sh, ssrc, sdst, sgain, nwin,
                rows, gsrc, gdst, ones, gsem, ssem, csem,
                counts, static_nwin, win=WIN):
    WIN = win

    def prep(w, b):
        for g in range(WIN // 16):
            gsrc[b, pl.ds(16 * g, 16)] = ssrc[pl.ds(WIN * w + 16 * g, 16)]
            gdst[b, pl.ds(16 * g, 16)] = sdst[pl.ds(WIN * w + 16 * g, 16)]
        pltpu.async_copy(h_hbm.at[gsrc.at[b]], rows.at[b], gsem.at[b])

    @pl.when(nwin >= 1)
    def _():
        prep(0, 0)

    def pair(t, _):
        for b in (0, 1):
            w = 2 * t + b
            nb = 1 - b

            @pl.when(w < nwin)
            def _():
                pltpu.make_async_copy(h_hbm.at[gsrc.at[b]], rows.at[b],
                                      gsem.at[b]).wait()

                @pl.when(w >= 1)
                def _():
                    pltpu.make_async_copy(rows.at[nb], acc_sh.at[gdst.at[nb]],
                                          ssem.at[nb]).wait()
                    if counts:
                        pltpu.make_async_copy(ones, cnt_sh.at[gdst.at[nb]],
                                              csem.at[nb]).wait()

                @pl.when(w + 1 < nwin)
                def _():
                    prep(w + 1, nb)

                def grp(i, _):
                    g16 = sgain[pl.ds(WIN * w + 16 * i, 16)]
                    for j in range(16):
                        e = 16 * i + j
                        g = g16[j]
                        for c in range(HD // 16):
                            rows[b, e, pl.ds(16 * c, 16)] = (
                                rows[b, e, pl.ds(16 * c, 16)] * g)
                    return 0
                lax.fori_loop(0, WIN // 16, grp, 0)
                pltpu.async_copy(rows.at[b], acc_sh.at[gdst.at[b]],
                                 ssem.at[b], add=True)
                if counts:
                    pltpu.async_copy(ones, cnt_sh.at[gdst.at[b]],
                                     csem.at[b], add=True)
        return 0
    lax.fori_loop(0, (nwin + 1) // 2, pair, 0)

    if static_nwin:
        b = (nwin - 1) % 2
        pltpu.make_async_copy(rows.at[b], acc_sh.at[gdst.at[b]],
                              ssem.at[b]).wait()
        if counts:
            pltpu.make_async_copy(ones, cnt_sh.at[gdst.at[b]],
                                  csem.at[b]).wait()
    else:
        for b in (0, 1):
            @pl.when((nwin >= 1) & ((nwin - 1) % 2 == b))
            def _(b=b):
                pltpu.make_async_copy(rows.at[b], acc_sh.at[gdst.at[b]],
                                      ssem.at[b]).wait()


def _gather_embeds(table_hbm, ids_hbm, out_hbm, idxb, rowsb, lsem, gsem, osem):
    cid = lax.axis_index("c")
    sid = lax.axis_index("s")
    wid = sid * 2 + cid
    share = GROWS // 32
    nwin = share // 128

    def load(w, b):
        pltpu.async_copy(ids_hbm.at[pl.ds(wid * share + w * 128, 128)],
                         idxb.at[b], lsem.at[b])

    load(0, 0)
    pltpu.make_async_copy(ids_hbm.at[pl.ds(wid * share, 128)],
                          idxb.at[0], lsem.at[0]).wait()
    pltpu.async_copy(table_hbm.at[idxb.at[0]], rowsb.at[0], gsem.at[0])
    load(1, 1)

    def pair(t, _):
        for b in (0, 1):
            w = 2 * t + b
            nb = 1 - b

            @pl.when(w < nwin)
            def _():
                base = wid * share + w * 128
                pltpu.make_async_copy(table_hbm.at[idxb.at[b]], rowsb.at[b],
                                      gsem.at[b]).wait()

                @pl.when(w + 1 < nwin)
                def _():
                    pltpu.make_async_copy(
                        ids_hbm.at[pl.ds(base, 128)], idxb.at[nb],
                        lsem.at[nb]).wait()

                    @pl.when(w >= 1)
                    def _():
                        pltpu.make_async_copy(
                            rowsb.at[nb], out_hbm.at[pl.ds(base, 128)],
                            osem.at[nb]).wait()
                    pltpu.async_copy(table_hbm.at[idxb.at[nb]], rowsb.at[nb],
                                     gsem.at[nb])

                    @pl.when(w + 2 < nwin)
                    def _():
                        load(w + 2, b)
                pltpu.async_copy(rowsb.at[b], out_hbm.at[pl.ds(base, 128)],
                                 osem.at[b])
        return 0
    lax.fori_loop(0, (nwin + 1) // 2, pair, 0)
    b = (nwin - 1) % 2
    pltpu.make_async_copy(rowsb.at[b],
                          out_hbm.at[pl.ds(wid * share, 128)], osem.at[b]).wait()
    @pl.when(nwin >= 2)
    def _():
        pltpu.make_async_copy(rowsb.at[1 - b],
                              out_hbm.at[pl.ds(wid * share, 128)],
                              osem.at[1 - b]).wait()


_gather_embeds = functools.partial(
    pl.kernel, mesh=_mesh, compiler_params=_params,
    out_type=jax.ShapeDtypeStruct((GROWS, HD), jnp.float32),
    scratch_types=[
        pltpu.VMEM((2, 128), jnp.int32),
        pltpu.VMEM((2, 128, HD), jnp.float32),
        pltpu.SemaphoreType.DMA((2,)),
        pltpu.SemaphoreType.DMA((2,)),
        pltpu.SemaphoreType.DMA((2,)),
    ],
)(_gather_embeds)


def _make_pair_agg(n_e1, n_e2, cacc, sch1, sch2):
    zlen = cacc // 16
    PW = 128
    stg = max(sch1, sch2)

    @functools.partial(
        pl.kernel, mesh=_mesh, compiler_params=_params,
        out_type=(jax.ShapeDtypeStruct((4, cacc, HD), jnp.float32),
                  jax.ShapeDtypeStruct((4 * cacc,), jnp.float32)),
        scratch_types=[
            pltpu.VMEM_SHARED((cacc, HD), jnp.float32),
            pltpu.VMEM_SHARED((cacc,), jnp.float32),
            pltpu.VMEM_SHARED((cacc,), jnp.float32),
            pltpu.VMEM((stg,), jnp.int32),
            pltpu.VMEM((stg,), jnp.int32),
            pltpu.VMEM((stg,), jnp.float32),
            pltpu.VMEM((PW,), jnp.float32),
            pltpu.VMEM((2, PW), jnp.int32),
            pltpu.VMEM((2, PW), jnp.int32),
            pltpu.VMEM((2, PW, HD), jnp.float32),
            pltpu.SemaphoreType.DMA((2,)),
            pltpu.SemaphoreType.DMA((2,)),
            pltpu.SemaphoreType.DMA((2,)),
        ],
    )
    def k(s1, d1, w1, ha, s2, d2, w2, hb, out_hbm, outc_hbm,
          acc, cntA, cntB, ssrc, sdst, sgain, ones,
          gsrc, gdst, rows, gsem, ssem, csem):
        cid = lax.axis_index("c")
        sid = lax.axis_index("s")
        wid = sid * 2 + cid
        _fill_ones_1d(ones, PW)
        _fill_zero_1d(sgain, stg)
        chunk = cacc // 16
        _zero_shared_1d(cntA, sgain, stg, sid * zlen, zlen)
        _zero_shared_1d(cntB, sgain, stg, sid * zlen, zlen)
        for ei, (src, dst, wgt, h, cnt, n_e, sch) in enumerate((
                (s1, d1, w1, ha, cntA, n_e1, sch1),
                (s2, d2, w2, hb, cntB, n_e2, sch2))):
            _fill_zero_2d(rows.at[0], PW)
            _zero_shared_rows(acc, rows.at[0], sid * chunk, chunk, PW)
            plsc.subcore_barrier()
            share = n_e // 32
            nch = share // sch

            def chunk_body(scn, _, src=src, dst=dst, wgt=wgt, h=h,
                           cnt=cnt, share=share, sch=sch):
                base = wid * share + scn * sch
                pltpu.sync_copy(src.at[pl.ds(base, sch)], ssrc.at[pl.ds(0, sch)])
                pltpu.sync_copy(dst.at[pl.ds(base, sch)], sdst.at[pl.ds(0, sch)])
                pltpu.sync_copy(wgt.at[pl.ds(base, sch)], sgain.at[pl.ds(0, sch)])
                _win_engine(h, acc, cnt, ssrc, sdst, sgain, sch // PW,
                            rows, gsrc, gdst, ones, gsem, ssem, csem,
                            True, True, win=PW)
                return 0
            lax.fori_loop(0, nch, chunk_body, 0)
            plsc.subcore_barrier()
            pltpu.sync_copy(acc.at[pl.ds(sid * chunk, chunk)],
                            out_hbm.at[2 * ei + cid, pl.ds(sid * chunk, chunk)])
            plsc.subcore_barrier()
        @pl.when(sid == 0)
        def _():
            pltpu.sync_copy(cntA, outc_hbm.at[pl.ds(cid * cacc, cacc)])
            pltpu.sync_copy(cntB, outc_hbm.at[pl.ds((2 + cid) * cacc, cacc)])
    return k


_agg_topic = _make_pair_agg(WT_E, TT_E, CT, 2560, 640)
_agg_doc = _make_pair_agg(WD_E, TD_E, CD, 2560, 1280)


WW_SHARE = WW_E // 16
WW_NCH = WW_SHARE // WW_SCH


@functools.partial(
    pl.kernel, mesh=_mesh, compiler_params=_params,
    out_type=(jax.ShapeDtypeStruct((CW, HD), jnp.float32),
              jax.ShapeDtypeStruct((CW,), jnp.float32)),
    scratch_types=[
        pltpu.VMEM_SHARED((WRANGE, HD), jnp.float32),
        pltpu.VMEM_SHARED((CW,), jnp.float32),
        pltpu.VMEM((WW_SCH,), jnp.int32),
        pltpu.VMEM((WW_SCH,), jnp.int32),
        pltpu.VMEM((WW_SCH,), jnp.float32),
        pltpu.VMEM((WW_CB,), jnp.int32),
        pltpu.VMEM((WW_CB,), jnp.int32),
        pltpu.VMEM((WW_CB,), jnp.float32),
        pltpu.VMEM((WIN,), jnp.float32),
        pltpu.VMEM((2, WIN), jnp.int32),
        pltpu.VMEM((2, WIN), jnp.int32),
        pltpu.VMEM((2, WIN), jnp.int32),
        pltpu.VMEM((2, WIN, HD), jnp.float32),
        pltpu.SemaphoreType.DMA((2,)),
        pltpu.SemaphoreType.DMA((2,)),
        pltpu.SemaphoreType.DMA((2,)),
    ],
)
def _agg_ww(src_hbm, dst_hbm, w_hbm, h_hbm, out_hbm, outc_hbm,
            acc_sh, cnt_sh, srcsc, dstsc, wsc, csrc, cdloc, cgain,
            ones, gsrc, gdst, cidx, rows, gsem, ssem, csem):
    cid = lax.axis_index("c")
    sid = lax.axis_index("s")
    wid = sid * 2 + cid
    _fill_ones_1d(ones, WIN)
    _fill_zero_1d(cgain, WW_CB)
    zc = CW // 16
    _zero_shared_1d(cnt_sh, cgain, WW_CB, sid * zc, zc)
    plsc.subcore_barrier()
    iota16 = lax.broadcasted_iota(jnp.int32, (16,), 0)
    chunk = WRANGE // 16
    for p in range(2):
        lo = (cid * 2 + p) * WRANGE
        _fill_zero_2d(rows.at[0], WIN)
        _zero_shared_rows(acc_sh, rows.at[0], sid * chunk, chunk)
        plsc.subcore_barrier()

        def schunk(scn, _):
            base = sid * WW_SHARE + scn * WW_SCH
            pltpu.sync_copy(src_hbm.at[pl.ds(base, WW_SCH)], srcsc)
            pltpu.sync_copy(dst_hbm.at[pl.ds(base, WW_SCH)], dstsc)
            pltpu.sync_copy(w_hbm.at[pl.ds(base, WW_SCH)], wsc)
            if p == 0:
                def cpair(t, _):
                    for b in (0, 1):
                        w = 2 * t + b

                        @pl.when(jnp.int32(w) >= 2)
                        def _():
                            pltpu.make_async_copy(
                                ones, cnt_sh.at[cidx.at[b]], csem.at[b]).wait()
                        for g in range(WIN // 16):
                            cidx[b, pl.ds(16 * g, 16)] = (
                                dstsc[pl.ds(WIN * w + 16 * g, 16)])
                        pltpu.async_copy(ones, cnt_sh.at[cidx.at[b]],
                                         csem.at[b], add=True)
                    return 0
                lax.fori_loop(0, WW_SCH // WIN // 2, cpair, 0)
                for b in (0, 1):
                    pltpu.make_async_copy(ones, cnt_sh.at[cidx.at[b]],
                                          csem.at[b]).wait()

            def compact(i, pos):
                d16 = dstsc[pl.ds(16 * i, 16)]
                w16 = wsc[pl.ds(16 * i, 16)]
                s16 = srcsc[pl.ds(16 * i, 16)]
                inr = (d16 >= lo) & (d16 < lo + WRANGE)
                plsc.store_compressed(csrc.at[pl.ds(pos, 16)], s16, mask=inr)
                plsc.store_compressed(cdloc.at[pl.ds(pos, 16)], d16 - lo, mask=inr)
                plsc.store_compressed(cgain.at[pl.ds(pos, 16)], w16, mask=inr)
                return pos + plsc.all_reduce_population_count(inr)[0]
            pos = lax.fori_loop(0, WW_SCH // 16, compact, jnp.int32(0))
            for kk in range(WIN // 16):
                pad_idx = iota16 + (16 * kk + wid * 128)
                csrc[pl.ds(pos + 16 * kk, 16)] = pad_idx
                cdloc[pl.ds(pos + 16 * kk, 16)] = jnp.zeros((16,), jnp.int32)
                cgain[pl.ds(pos + 16 * kk, 16)] = jnp.zeros((16,), jnp.float32)
            _win_engine(h_hbm, acc_sh, None, csrc, cdloc, cgain,
                        (pos + WIN - 1) // WIN,
                        rows, gsrc, gdst, ones, gsem, ssem, csem,
                        False, False)
            return 0
        lax.fori_loop(0, WW_NCH, schunk, 0)
        plsc.subcore_barrier()
        pltpu.sync_copy(acc_sh.at[pl.ds(sid * chunk, chunk)],
                        out_hbm.at[pl.ds(lo + sid * chunk, chunk)])
        if p == 0:
            @pl.when((cid == 0) & (sid == 0))
            def _():
                pltpu.sync_copy(cnt_sh, outc_hbm)
        plsc.subcore_barrier()


def _prep_body(aW, ab, cW, cb, tids, temb,
               m0_o, b0_o, m1_o, b1_o, t1_o, tb1_o, ht0_o):
    m0 = aW[...]
    b0 = ab[...]
    for i in range(3):
        w = cW[0, i]
        m0 = jnp.dot(m0, w, preferred_element_type=jnp.float32)
        b0 = jnp.dot(b0, w, preferred_element_type=jnp.float32) + cb[0, i]
    m1 = cW[1, 0]
    b1 = cb[1, 0]
    for i in range(1, 3):
        w = cW[1, i]
        m1 = jnp.dot(m1, w, preferred_element_type=jnp.float32)
        b1 = jnp.dot(b1, w, preferred_element_type=jnp.float32) + cb[1, i]
    t1 = jnp.dot(cW[1, 3], cW[1, 4], preferred_element_type=jnp.float32)
    tb1 = jnp.dot(cb[1, 3], cW[1, 4], preferred_element_type=jnp.float32) + cb[1, 4]
    tt0 = jnp.dot(temb[...], cW[0, 3], preferred_element_type=jnp.float32) + cb[0, 3]
    tt0 = jnp.dot(tt0, cW[0, 4], preferred_element_type=jnp.float32) + cb[0, 4]
    ids = tids[...]
    oh = (lax.broadcasted_iota(jnp.int32, (NTN, 64), 1) == ids).astype(jnp.float32)
    ht0_o[...] = jnp.dot(oh, tt0, preferred_element_type=jnp.float32)
    m0_o[...] = m0
    b0_o[...] = b0
    m1_o[...] = m1
    b1_o[...] = b1
    t1_o[...] = t1
    tb1_o[...] = tb1


def _prep(aW, ab, cW, cb, tids, temb):
    return pl.pallas_call(
        _prep_body,
        out_shape=(
            jax.ShapeDtypeStruct((HI, HD), jnp.float32),
            jax.ShapeDtypeStruct((1, HD), jnp.float32),
            jax.ShapeDtypeStruct((HD, HD), jnp.float32),
            jax.ShapeDtypeStruct((1, HD), jnp.float32),
            jax.ShapeDtypeStruct((HD, HD), jnp.float32),
            jax.ShapeDtypeStruct((1, HD), jnp.float32),
            jax.ShapeDtypeStruct((NTN, HD), jnp.float32),
        ),
    )(aW, ab, cW, cb, tids, temb)


def _matmul_bias(x, m, b, tr=512):
    n, k = x.shape
    assert n % tr == 0

    def body(x_ref, m_ref, b_ref, o_ref):
        o_ref[...] = jnp.dot(x_ref[...], m_ref[...],
                             preferred_element_type=jnp.float32) + b_ref[...]

    return pl.pallas_call(
        body,
        grid=(n // tr,),
        in_specs=[
            pl.BlockSpec((tr, k), lambda i: (i, 0)),
            pl.BlockSpec((k, HD), lambda i: (0, 0)),
            pl.BlockSpec((1, HD), lambda i: (0, 0)),
        ],
        out_specs=pl.BlockSpec((tr, HD), lambda i: (i, 0)),
        out_shape=jax.ShapeDtypeStruct((n, HD), jnp.float32),
    )(x, m, b)


def _scale_relu_matmul(x, c, m, b, tr=512):
    n, k = x.shape
    assert n % tr == 0

    def body(x_ref, c_ref, m_ref, b_ref, o_ref):
        r = 1.0 / jnp.maximum(c_ref[...], 1.0)
        h = jnp.maximum(x_ref[...], 0.0) * r
        o_ref[...] = jnp.dot(h, m_ref[...],
                             preferred_element_type=jnp.float32) + b_ref[...]

    return pl.pallas_call(
        body,
        grid=(n // tr,),
        in_specs=[
            pl.BlockSpec((tr, k), lambda i: (i, 0)),
            pl.BlockSpec((tr, 1), lambda i: (i, 0)),
            pl.BlockSpec((k, HD), lambda i: (0, 0)),
            pl.BlockSpec((1, HD), lambda i: (0, 0)),
        ],
        out_specs=pl.BlockSpec((tr, HD), lambda i: (i, 0)),
        out_shape=jax.ShapeDtypeStruct((n, HD), jnp.float32),
    )(x, c, m, b)


def _combine_pair_matmul(acc, cnt, m, b):
    _, n, _ = acc.shape

    def body(a_ref, c_ref, m_ref, b_ref, o_ref):
        ra = 1.0 / jnp.maximum(c_ref[0] + c_ref[1], 1.0)
        rb = 1.0 / jnp.maximum(c_ref[2] + c_ref[3], 1.0)
        h = jnp.maximum((a_ref[0] + a_ref[1]) * ra
                        + (a_ref[2] + a_ref[3]) * rb, 0.0)
        o_ref[...] = jnp.dot(h, m_ref[...],
                             preferred_element_type=jnp.float32) + b_ref[...]

    return pl.pallas_call(
        body,
        out_shape=jax.ShapeDtypeStruct((n, HD), jnp.float32),
    )(acc, cnt, m, b)


def _final(acc, cnt, ow, ob, y):
    def body(a_ref, c_ref, ow_ref, ob_ref, y_ref, loss_ref, yp_ref):
        ra = 1.0 / jnp.maximum(c_ref[0] + c_ref[1], 1.0)
        rb = 1.0 / jnp.maximum(c_ref[2] + c_ref[3], 1.0)
        hd = jnp.maximum((a_ref[0] + a_ref[1]) * ra
                         + (a_ref[2] + a_ref[3]) * rb, 0.0)
        ms = []
        for g in range(NB):
            ms.append(jnp.max(hd[100 * g:100 * g + 100, :], axis=0,
                              keepdims=True))
        glob = jnp.concatenate(ms, axis=0)
        z = jnp.sum(glob * ow_ref[...], axis=1, keepdims=True) + ob_ref[...]
        yv = y_ref[...]
        lossv = jnp.mean(jnp.maximum(z, 0.0) - z * yv
                         + jnp.log(1.0 + jnp.exp(-jnp.abs(z))))
        loss_ref[...] = lossv[None, None]
        yp_ref[...] = 1.0 / (1.0 + jnp.exp(-z))

    return pl.pallas_call(
        body,
        out_shape=(
            jax.ShapeDtypeStruct((1, 1), jnp.float32),
            jax.ShapeDtypeStruct((NB, 1), jnp.float32),
        ),
    )(acc, cnt, ow, ob, y)


def _pad_edges(src, dst, w, n_pad, n_src, n_dst):
    e = src.shape[0]
    k = n_pad - e
    pad_src = (jnp.arange(k, dtype=jnp.int32) * 7919) % n_src
    src = jnp.concatenate([src.astype(jnp.int32), pad_src])
    dst = jnp.concatenate([dst.astype(jnp.int32),
                           jnp.full((k,), n_dst, jnp.int32)])
    w = jnp.concatenate([w, jnp.zeros((k,), w.dtype)])
    return src, dst, w


def kernel(word_ids, topic_ids, ww_src, ww_dst, ww_w, wt_src, wt_dst, wt_w,
           wd_src, wd_dst, wd_w, td_src, td_dst, td_w, tt_src, tt_dst, tt_w,
           doc_graph_ids, y_data, word_embeds, topic_embeds, adapt_W, adapt_b,
           conv_W, conv_b, out_W, out_b):
    wid_pad = jnp.concatenate([
        word_ids.astype(jnp.int32),
        (jnp.arange(GROWS - NWN, dtype=jnp.int32) * 7919) % VOC])
    ww = _pad_edges(ww_src, ww_dst, ww_w, WW_E, NWN, NWN)
    wt = _pad_edges(wt_src, wt_dst, wt_w, WT_E, NWN, NTN)
    wd = _pad_edges(wd_src, wd_dst, wd_w, WD_E, NWN, NDN)
    td = _pad_edges(td_src, td_dst, td_w, TD_E, NTN, NDN)
    tt = _pad_edges(tt_src, tt_dst, tt_w, TT_E, NTN, NTN)
    temb_pad = jnp.concatenate(
        [topic_embeds, jnp.zeros((14, HD), jnp.float32)], axis=0)
    cb4 = conv_b.reshape(2, 5, 1, HD)
    ab2 = adapt_b.reshape(1, HD)
    tids2 = topic_ids.astype(jnp.int32).reshape(NTN, 1)
    y2 = y_data.reshape(NB, 1)
    ow2 = out_W.reshape(1, HD)
    ob2 = out_b.reshape(1, 1)

    m0, b0, m1, b1, t1, tb1, ht0 = _prep(adapt_W, ab2, conv_W, cb4, tids2, temb_pad)

    tword = _matmul_bias(word_embeds, m0, b0, tr=1000)
    hw0 = _gather_embeds(tword, wid_pad)

    at, atc = _agg_topic(wt[0], wt[1], wt[2], hw0, tt[0], tt[1], tt[2], ht0)
    aww, awwc = _agg_ww(ww[0], ww[1], ww[2], hw0)

    hw1 = _scale_relu_matmul(aww, awwc.reshape(CW, 1), m1, b1)
    ht1 = _combine_pair_matmul(at, atc.reshape(4, CT, 1), t1, tb1)

    ad, adc = _agg_doc(wd[0], wd[1], wd[2], hw1, td[0], td[1], td[2], ht1)

    loss2, yp = _final(ad, adc.reshape(4, CD, 1), ow2, ob2, y2)
    return (loss2.reshape(()), yp)


# speedup vs baseline: 4.0746x; 1.0363x over previous
import functools

import jax
import jax.numpy as jnp
from jax import lax
from jax.experimental import pallas as pl
from jax.experimental.pallas import tpu as pltpu
from jax.experimental.pallas import tpu_sc as plsc

HD = 128
NWN = 50000
NTN = 3200
NDN = 6400
NB = 64
VOC = 100000
HI = 300

WW_E = 327680
WT_E, WD_E, TD_E, TT_E = 81920, 163840, 40960, 20480
GROWS = 53248
CW = 50176
WRANGE = 12544
CT = 3328
CD = 6656
WIN = 64
STG = 1280
WW_SCH = 1024
WW_CB = 1088

_mesh = plsc.VectorSubcoreMesh(core_axis_name="c", subcore_axis_name="s")
_params = pltpu.CompilerParams(needs_layout_passes=False)


def _fill_zero_2d(ref, nrows):
    def body(i, _):
        for c in range(HD // 16):
            ref[i, pl.ds(16 * c, 16)] = jnp.zeros((16,), jnp.float32)
        return 0
    lax.fori_loop(0, nrows, body, 0)


def _fill_zero_1d(ref, n):
    def body(i, _):
        ref[pl.ds(16 * i, 16)] = jnp.zeros((16,), jnp.float32)
        return 0
    lax.fori_loop(0, n // 16, body, 0)


def _fill_ones_1d(ref, n):
    for i in range(n // 16):
        ref[pl.ds(16 * i, 16)] = jnp.ones((16,), jnp.float32)


def _zero_shared_rows(acc_sh, zrows, row0, nrows, win=WIN):
    off = 0
    while off < nrows:
        n = min(win, nrows - off)
        pltpu.sync_copy(zrows.at[pl.ds(0, n)], acc_sh.at[pl.ds(row0 + off, n)])
        off += n


def _zero_shared_1d(cnt_sh, zbuf, zlen, start, total):
    off = 0
    while off < total:
        n = min(zlen - (zlen % 8), total - off)
        n = n - (n % 8)
        pltpu.sync_copy(zbuf.at[pl.ds(0, n)], cnt_sh.at[pl.ds(start + off, n)])
        off += n


def _win_engine(h_hbm, acc_sh, cnt_sh, ssrc, sdst, sgain, nwin,
                rows, gsrc, gdst, ones, gsem, ssem, csem,
                counts, static_nwin, win=WIN):
    WIN = win

    def prep(w, b):
        for g in range(WIN // 16):
            gsrc[b, pl.ds(16 * g, 16)] = ssrc[pl.ds(WIN * w + 16 * g, 16)]
            gdst[b, pl.ds(16 * g, 16)] = sdst[pl.ds(WIN * w + 16 * g, 16)]
        pltpu.async_copy(h_hbm.at[gsrc.at[b]], rows.at[b], gsem.at[b])

    @pl.when(nwin >= 1)
    def _():
        prep(0, 0)

    def pair(t, _):
        for b in (0, 1):
            w = 2 * t + b
            nb = 1 - b

            @pl.when(w < nwin)
            def _():
                pltpu.make_async_copy(h_hbm.at[gsrc.at[b]], rows.at[b],
                                      gsem.at[b]).wait()

                @pl.when(w >= 1)
                def _():
                    pltpu.make_async_copy(rows.at[nb], acc_sh.at[gdst.at[nb]],
                                          ssem.at[nb]).wait()
                    if counts:
                        pltpu.make_async_copy(ones, cnt_sh.at[gdst.at[nb]],
                                              csem.at[nb]).wait()

                @pl.when(w + 1 < nwin)
                def _():
                    prep(w + 1, nb)

                def grp(i, _):
                    g16 = sgain[pl.ds(WIN * w + 16 * i, 16)]
                    for j in range(16):
                        e = 16 * i + j
                        g = g16[j]
                        for c in range(HD // 16):
                            rows[b, e, pl.ds(16 * c, 16)] = (
                                rows[b, e, pl.ds(16 * c, 16)] * g)
                    return 0
                lax.fori_loop(0, WIN // 16, grp, 0)
                pltpu.async_copy(rows.at[b], acc_sh.at[gdst.at[b]],
                                 ssem.at[b], add=True)
                if counts:
                    pltpu.async_copy(ones, cnt_sh.at[gdst.at[b]],
                                     csem.at[b], add=True)
        return 0
    lax.fori_loop(0, (nwin + 1) // 2, pair, 0)

    if static_nwin:
        b = (nwin - 1) % 2
        pltpu.make_async_copy(rows.at[b], acc_sh.at[gdst.at[b]],
                              ssem.at[b]).wait()
        if counts:
            pltpu.make_async_copy(ones, cnt_sh.at[gdst.at[b]],
                                  csem.at[b]).wait()
    else:
        for b in (0, 1):
            @pl.when((nwin >= 1) & ((nwin - 1) % 2 == b))
            def _(b=b):
                pltpu.make_async_copy(rows.at[b], acc_sh.at[gdst.at[b]],
                                      ssem.at[b]).wait()


def _gather_embeds(table_hbm, ids_hbm, out_hbm, idxb, rowsb, lsem, gsem, osem):
    cid = lax.axis_index("c")
    sid = lax.axis_index("s")
    wid = sid * 2 + cid
    share = GROWS // 32
    nwin = share // 128

    def load(w, b):
        pltpu.async_copy(ids_hbm.at[pl.ds(wid * share + w * 128, 128)],
                         idxb.at[b], lsem.at[b])

    load(0, 0)
    pltpu.make_async_copy(ids_hbm.at[pl.ds(wid * share, 128)],
                          idxb.at[0], lsem.at[0]).wait()
    pltpu.async_copy(table_hbm.at[idxb.at[0]], rowsb.at[0], gsem.at[0])
    load(1, 1)

    def pair(t, _):
        for b in (0, 1):
            w = 2 * t + b
            nb = 1 - b

            @pl.when(w < nwin)
            def _():
                base = wid * share + w * 128
                pltpu.make_async_copy(table_hbm.at[idxb.at[b]], rowsb.at[b],
                                      gsem.at[b]).wait()

                @pl.when(w + 1 < nwin)
                def _():
                    pltpu.make_async_copy(
                        ids_hbm.at[pl.ds(base, 128)], idxb.at[nb],
                        lsem.at[nb]).wait()

                    @pl.when(w >= 1)
                    def _():
                        pltpu.make_async_copy(
                            rowsb.at[nb], out_hbm.at[pl.ds(base, 128)],
                            osem.at[nb]).wait()
                    pltpu.async_copy(table_hbm.at[idxb.at[nb]], rowsb.at[nb],
                                     gsem.at[nb])

                    @pl.when(w + 2 < nwin)
                    def _():
                        load(w + 2, b)
                pltpu.async_copy(rowsb.at[b], out_hbm.at[pl.ds(base, 128)],
                                 osem.at[b])
        return 0
    lax.fori_loop(0, (nwin + 1) // 2, pair, 0)
    b = (nwin - 1) % 2
    pltpu.make_async_copy(rowsb.at[b],
                          out_hbm.at[pl.ds(wid * share, 128)], osem.at[b]).wait()
    @pl.when(nwin >= 2)
    def _():
        pltpu.make_async_copy(rowsb.at[1 - b],
                              out_hbm.at[pl.ds(wid * share, 128)],
                              osem.at[1 - b]).wait()


_gather_embeds = functools.partial(
    pl.kernel, mesh=_mesh, compiler_params=_params,
    out_type=jax.ShapeDtypeStruct((GROWS, HD), jnp.float32),
    scratch_types=[
        pltpu.VMEM((2, 128), jnp.int32),
        pltpu.VMEM((2, 128, HD), jnp.float32),
        pltpu.SemaphoreType.DMA((2,)),
        pltpu.SemaphoreType.DMA((2,)),
        pltpu.SemaphoreType.DMA((2,)),
    ],
)(_gather_embeds)


def _make_pair_agg(n_e1, n_e2, cacc, sch1, sch2):
    zlen = cacc // 16
    PW = 128
    stg = max(sch1, sch2)

    @functools.partial(
        pl.kernel, mesh=_mesh, compiler_params=_params,
        out_type=(jax.ShapeDtypeStruct((4, cacc, HD), jnp.float32),
                  jax.ShapeDtypeStruct((4 * cacc,), jnp.float32)),
        scratch_types=[
            pltpu.VMEM_SHARED((cacc, HD), jnp.float32),
            pltpu.VMEM_SHARED((cacc,), jnp.float32),
            pltpu.VMEM_SHARED((cacc,), jnp.float32),
            pltpu.VMEM((stg,), jnp.int32),
            pltpu.VMEM((stg,), jnp.int32),
            pltpu.VMEM((stg,), jnp.float32),
            pltpu.VMEM((PW,), jnp.float32),
            pltpu.VMEM((2, PW), jnp.int32),
            pltpu.VMEM((2, PW), jnp.int32),
            pltpu.VMEM((2, PW, HD), jnp.float32),
            pltpu.SemaphoreType.DMA((2,)),
            pltpu.SemaphoreType.DMA((2,)),
            pltpu.SemaphoreType.DMA((2,)),
        ],
    )
    def k(s1, d1, w1, ha, s2, d2, w2, hb, out_hbm, outc_hbm,
          acc, cntA, cntB, ssrc, sdst, sgain, ones,
          gsrc, gdst, rows, gsem, ssem, csem):
        cid = lax.axis_index("c")
        sid = lax.axis_index("s")
        wid = sid * 2 + cid
        _fill_ones_1d(ones, PW)
        _fill_zero_1d(sgain, stg)
        chunk = cacc // 16
        _zero_shared_1d(cntA, sgain, stg, sid * zlen, zlen)
        _zero_shared_1d(cntB, sgain, stg, sid * zlen, zlen)
        for ei, (src, dst, wgt, h, cnt, n_e, sch) in enumerate((
                (s1, d1, w1, ha, cntA, n_e1, sch1),
                (s2, d2, w2, hb, cntB, n_e2, sch2))):
            _fill_zero_2d(rows.at[0], PW)
            _zero_shared_rows(acc, rows.at[0], sid * chunk, chunk, PW)
            plsc.subcore_barrier()
            share = n_e // 32
            nch = share // sch

            def chunk_body(scn, _, src=src, dst=dst, wgt=wgt, h=h,
                           cnt=cnt, share=share, sch=sch):
                base = wid * share + scn * sch
                pltpu.sync_copy(src.at[pl.ds(base, sch)], ssrc.at[pl.ds(0, sch)])
                pltpu.sync_copy(dst.at[pl.ds(base, sch)], sdst.at[pl.ds(0, sch)])
                pltpu.sync_copy(wgt.at[pl.ds(base, sch)], sgain.at[pl.ds(0, sch)])
                _win_engine(h, acc, cnt, ssrc, sdst, sgain, sch // PW,
                            rows, gsrc, gdst, ones, gsem, ssem, csem,
                            True, True, win=PW)
                return 0
            lax.fori_loop(0, nch, chunk_body, 0)
            plsc.subcore_barrier()
            pltpu.sync_copy(acc.at[pl.ds(sid * chunk, chunk)],
                            out_hbm.at[2 * ei + cid, pl.ds(sid * chunk, chunk)])
            plsc.subcore_barrier()
        @pl.when(sid == 0)
        def _():
            pltpu.sync_copy(cntA, outc_hbm.at[pl.ds(cid * cacc, cacc)])
            pltpu.sync_copy(cntB, outc_hbm.at[pl.ds((2 + cid) * cacc, cacc)])
    return k


_agg_topic = _make_pair_agg(WT_E, TT_E, CT, 2560, 640)
_agg_doc = _make_pair_agg(WD_E, TD_E, CD, 2560, 1280)


WW_SHARE = WW_E // 16
WW_NCH = WW_SHARE // WW_SCH


@functools.partial(
    pl.kernel, mesh=_mesh, compiler_params=_params,
    out_type=(jax.ShapeDtypeStruct((CW, HD), jnp.float32),
              jax.ShapeDtypeStruct((CW,), jnp.float32)),
    scratch_types=[
        pltpu.VMEM_SHARED((WRANGE, HD), jnp.float32),
        pltpu.VMEM_SHARED((CW,), jnp.float32),
        pltpu.VMEM((2, WW_SCH), jnp.int32),
        pltpu.VMEM((2, WW_SCH), jnp.int32),
        pltpu.VMEM((2, WW_SCH), jnp.float32),
        pltpu.VMEM((WW_CB,), jnp.int32),
        pltpu.VMEM((WW_CB,), jnp.int32),
        pltpu.VMEM((WW_CB,), jnp.float32),
        pltpu.VMEM((WIN,), jnp.float32),
        pltpu.VMEM((2, WIN), jnp.int32),
        pltpu.VMEM((2, WIN), jnp.int32),
        pltpu.VMEM((2, WIN), jnp.int32),
        pltpu.VMEM((2, WIN, HD), jnp.float32),
        pltpu.SemaphoreType.DMA((2,)),
        pltpu.SemaphoreType.DMA((2,)),
        pltpu.SemaphoreType.DMA((2,)),
        pltpu.SemaphoreType.DMA((2,)),
    ],
)
def _agg_ww(src_hbm, dst_hbm, w_hbm, h_hbm, out_hbm, outc_hbm,
            acc_sh, cnt_sh, srcsc, dstsc, wsc, csrc, cdloc, cgain,
            ones, gsrc, gdst, cidx, rows, gsem, ssem, csem, stsem):
    cid = lax.axis_index("c")
    sid = lax.axis_index("s")
    wid = sid * 2 + cid
    _fill_ones_1d(ones, WIN)
    _fill_zero_1d(cgain, WW_CB)
    zc = CW // 16
    _zero_shared_1d(cnt_sh, cgain, WW_CB, sid * zc, zc)
    plsc.subcore_barrier()
    iota16 = lax.broadcasted_iota(jnp.int32, (16,), 0)
    chunk = WRANGE // 16

    def stage(scn, sl):
        base = sid * WW_SHARE + scn * WW_SCH
        pltpu.async_copy(src_hbm.at[pl.ds(base, WW_SCH)], srcsc.at[sl],
                         stsem.at[sl])
        pltpu.async_copy(dst_hbm.at[pl.ds(base, WW_SCH)], dstsc.at[sl],
                         stsem.at[sl])
        pltpu.async_copy(w_hbm.at[pl.ds(base, WW_SCH)], wsc.at[sl],
                         stsem.at[sl])

    def stage_wait(sl):
        base = sid * WW_SHARE
        pltpu.make_async_copy(src_hbm.at[pl.ds(base, WW_SCH)], srcsc.at[sl],
                              stsem.at[sl]).wait()
        pltpu.make_async_copy(dst_hbm.at[pl.ds(base, WW_SCH)], dstsc.at[sl],
                              stsem.at[sl]).wait()
        pltpu.make_async_copy(w_hbm.at[pl.ds(base, WW_SCH)], wsc.at[sl],
                              stsem.at[sl]).wait()

    for p in range(2):
        lo = (cid * 2 + p) * WRANGE
        _fill_zero_2d(rows.at[0], WIN)
        _zero_shared_rows(acc_sh, rows.at[0], sid * chunk, chunk)
        plsc.subcore_barrier()
        stage(0, 0)

        def schunk_pair(t, _):
            for par in (0, 1):
                scn = 2 * t + par
                npar = 1 - par
                stage_wait(par)

                @pl.when(scn + 1 < WW_NCH)
                def _():
                    stage(scn + 1, npar)
                if p == 0:
                    def cpair(tc, _):
                        for b in (0, 1):
                            w = 2 * tc + b

                            @pl.when(jnp.int32(w) >= 2)
                            def _():
                                pltpu.make_async_copy(
                                    ones, cnt_sh.at[cidx.at[b]],
                                    csem.at[b]).wait()
                            for g in range(WIN // 16):
                                cidx[b, pl.ds(16 * g, 16)] = (
                                    dstsc[par, pl.ds(WIN * w + 16 * g, 16)])
                            pltpu.async_copy(ones, cnt_sh.at[cidx.at[b]],
                                             csem.at[b], add=True)
                        return 0
                    lax.fori_loop(0, WW_SCH // WIN // 2, cpair, 0)
                    for b in (0, 1):
                        pltpu.make_async_copy(ones, cnt_sh.at[cidx.at[b]],
                                              csem.at[b]).wait()

                def compact(i, pos):
                    d16 = dstsc[par, pl.ds(16 * i, 16)]
                    w16 = wsc[par, pl.ds(16 * i, 16)]
                    s16 = srcsc[par, pl.ds(16 * i, 16)]
                    inr = (d16 >= lo) & (d16 < lo + WRANGE)
                    plsc.store_compressed(csrc.at[pl.ds(pos, 16)], s16,
                                          mask=inr)
                    plsc.store_compressed(cdloc.at[pl.ds(pos, 16)], d16 - lo,
                                          mask=inr)
                    plsc.store_compressed(cgain.at[pl.ds(pos, 16)], w16,
                                          mask=inr)
                    return pos + plsc.all_reduce_population_count(inr)[0]
                pos = lax.fori_loop(0, WW_SCH // 16, compact, jnp.int32(0))
                for kk in range(WIN // 16):
                    pad_idx = iota16 + (16 * kk + wid * 128)
                    csrc[pl.ds(pos + 16 * kk, 16)] = pad_idx
                    cdloc[pl.ds(pos + 16 * kk, 16)] = jnp.zeros((16,), jnp.int32)
                    cgain[pl.ds(pos + 16 * kk, 16)] = jnp.zeros((16,),
                                                               jnp.float32)
                _win_engine(h_hbm, acc_sh, None, csrc, cdloc, cgain,
                            (pos + WIN - 1) // WIN,
                            rows, gsrc, gdst, ones, gsem, ssem, csem,
                            False, False)
            return 0
        lax.fori_loop(0, WW_NCH // 2, schunk_pair, 0)
        plsc.subcore_barrier()
        pltpu.sync_copy(acc_sh.at[pl.ds(sid * chunk, chunk)],
                        out_hbm.at[pl.ds(lo + sid * chunk, chunk)])
        if p == 0:
            @pl.when((cid == 0) & (sid == 0))
            def _():
                pltpu.sync_copy(cnt_sh, outc_hbm)
        plsc.subcore_barrier()


def _prep_body(aW, ab, cW, cb, tids, temb,
               m0_o, b0_o, m1_o, b1_o, t1_o, tb1_o, ht0_o):
    m0 = aW[...]
    b0 = ab[...]
    for i in range(3):
        w = cW[0, i]
        m0 = jnp.dot(m0, w, preferred_element_type=jnp.float32)
        b0 = jnp.dot(b0, w, preferred_element_type=jnp.float32) + cb[0, i]
    m1 = cW[1, 0]
    b1 = cb[1, 0]
    for i in range(1, 3):
        w = cW[1, i]
        m1 = jnp.dot(m1, w, preferred_element_type=jnp.float32)
        b1 = jnp.dot(b1, w, preferred_element_type=jnp.float32) + cb[1, i]
    t1 = jnp.dot(cW[1, 3], cW[1, 4], preferred_element_type=jnp.float32)
    tb1 = jnp.dot(cb[1, 3], cW[1, 4], preferred_element_type=jnp.float32) + cb[1, 4]
    tt0 = jnp.dot(temb[...], cW[0, 3], preferred_element_type=jnp.float32) + cb[0, 3]
    tt0 = jnp.dot(tt0, cW[0, 4], preferred_element_type=jnp.float32) + cb[0, 4]
    ids = tids[...]
    oh = (lax.broadcasted_iota(jnp.int32, (NTN, 64), 1) == ids).astype(jnp.float32)
    ht0_o[...] = jnp.dot(oh, tt0, preferred_element_type=jnp.float32)
    m0_o[...] = m0
    b0_o[...] = b0
    m1_o[...] = m1
    b1_o[...] = b1
    t1_o[...] = t1
    tb1_o[...] = tb1


def _prep(aW, ab, cW, cb, tids, temb):
    return pl.pallas_call(
        _prep_body,
        out_shape=(
            jax.ShapeDtypeStruct((HI, HD), jnp.float32),
            jax.ShapeDtypeStruct((1, HD), jnp.float32),
            jax.ShapeDtypeStruct((HD, HD), jnp.float32),
            jax.ShapeDtypeStruct((1, HD), jnp.float32),
            jax.ShapeDtypeStruct((HD, HD), jnp.float32),
            jax.ShapeDtypeStruct((1, HD), jnp.float32),
            jax.ShapeDtypeStruct((NTN, HD), jnp.float32),
        ),
    )(aW, ab, cW, cb, tids, temb)


def _matmul_bias(x, m, b, tr=512):
    n, k = x.shape
    assert n % tr == 0

    def body(x_ref, m_ref, b_ref, o_ref):
        o_ref[...] = jnp.dot(x_ref[...], m_ref[...],
                             preferred_element_type=jnp.float32) + b_ref[...]

    return pl.pallas_call(
        body,
        grid=(n // tr,),
        in_specs=[
            pl.BlockSpec((tr, k), lambda i: (i, 0)),
            pl.BlockSpec((k, HD), lambda i: (0, 0)),
            pl.BlockSpec((1, HD), lambda i: (0, 0)),
        ],
        out_specs=pl.BlockSpec((tr, HD), lambda i: (i, 0)),
        out_shape=jax.ShapeDtypeStruct((n, HD), jnp.float32),
    )(x, m, b)


def _scale_relu_matmul(x, c, m, b, tr=512):
    n, k = x.shape
    assert n % tr == 0

    def body(x_ref, c_ref, m_ref, b_ref, o_ref):
        r = 1.0 / jnp.maximum(c_ref[...], 1.0)
        h = jnp.maximum(x_ref[...], 0.0) * r
        o_ref[...] = jnp.dot(h, m_ref[...],
                             preferred_element_type=jnp.float32) + b_ref[...]

    return pl.pallas_call(
        body,
        grid=(n // tr,),
        in_specs=[
            pl.BlockSpec((tr, k), lambda i: (i, 0)),
            pl.BlockSpec((tr, 1), lambda i: (i, 0)),
            pl.BlockSpec((k, HD), lambda i: (0, 0)),
            pl.BlockSpec((1, HD), lambda i: (0, 0)),
        ],
        out_specs=pl.BlockSpec((tr, HD), lambda i: (i, 0)),
        out_shape=jax.ShapeDtypeStruct((n, HD), jnp.float32),
    )(x, c, m, b)


def _combine_pair_matmul(acc, cnt, m, b):
    _, n, _ = acc.shape

    def body(a_ref, c_ref, m_ref, b_ref, o_ref):
        ra = 1.0 / jnp.maximum(c_ref[0] + c_ref[1], 1.0)
        rb = 1.0 / jnp.maximum(c_ref[2] + c_ref[3], 1.0)
        h = jnp.maximum((a_ref[0] + a_ref[1]) * ra
                        + (a_ref[2] + a_ref[3]) * rb, 0.0)
        o_ref[...] = jnp.dot(h, m_ref[...],
                             preferred_element_type=jnp.float32) + b_ref[...]

    return pl.pallas_call(
        body,
        out_shape=jax.ShapeDtypeStruct((n, HD), jnp.float32),
    )(acc, cnt, m, b)


def _final(acc, cnt, ow, ob, y):
    def body(a_ref, c_ref, ow_ref, ob_ref, y_ref, loss_ref, yp_ref):
        ra = 1.0 / jnp.maximum(c_ref[0] + c_ref[1], 1.0)
        rb = 1.0 / jnp.maximum(c_ref[2] + c_ref[3], 1.0)
        hd = jnp.maximum((a_ref[0] + a_ref[1]) * ra
                         + (a_ref[2] + a_ref[3]) * rb, 0.0)
        ms = []
        for g in range(NB):
            ms.append(jnp.max(hd[100 * g:100 * g + 100, :], axis=0,
                              keepdims=True))
        glob = jnp.concatenate(ms, axis=0)
        z = jnp.sum(glob * ow_ref[...], axis=1, keepdims=True) + ob_ref[...]
        yv = y_ref[...]
        lossv = jnp.mean(jnp.maximum(z, 0.0) - z * yv
                         + jnp.log(1.0 + jnp.exp(-jnp.abs(z))))
        loss_ref[...] = lossv[None, None]
        yp_ref[...] = 1.0 / (1.0 + jnp.exp(-z))

    return pl.pallas_call(
        body,
        out_shape=(
            jax.ShapeDtypeStruct((1, 1), jnp.float32),
            jax.ShapeDtypeStruct((NB, 1), jnp.float32),
        ),
    )(acc, cnt, ow, ob, y)


def _pad_edges(src, dst, w, n_pad, n_src, n_dst):
    e = src.shape[0]
    k = n_pad - e
    pad_src = (jnp.arange(k, dtype=jnp.int32) * 7919) % n_src
    src = jnp.concatenate([src.astype(jnp.int32), pad_src])
    dst = jnp.concatenate([dst.astype(jnp.int32),
                           jnp.full((k,), n_dst, jnp.int32)])
    w = jnp.concatenate([w, jnp.zeros((k,), w.dtype)])
    return src, dst, w


def kernel(word_ids, topic_ids, ww_src, ww_dst, ww_w, wt_src, wt_dst, wt_w,
           wd_src, wd_dst, wd_w, td_src, td_dst, td_w, tt_src, tt_dst, tt_w,
           doc_graph_ids, y_data, word_embeds, topic_embeds, adapt_W, adapt_b,
           conv_W, conv_b, out_W, out_b):
    wid_pad = jnp.concatenate([
        word_ids.astype(jnp.int32),
        (jnp.arange(GROWS - NWN, dtype=jnp.int32) * 7919) % VOC])
    ww = _pad_edges(ww_src, ww_dst, ww_w, WW_E, NWN, NWN)
    wt = _pad_edges(wt_src, wt_dst, wt_w, WT_E, NWN, NTN)
    wd = _pad_edges(wd_src, wd_dst, wd_w, WD_E, NWN, NDN)
    td = _pad_edges(td_src, td_dst, td_w, TD_E, NTN, NDN)
    tt = _pad_edges(tt_src, tt_dst, tt_w, TT_E, NTN, NTN)
    temb_pad = jnp.concatenate(
        [topic_embeds, jnp.zeros((14, HD), jnp.float32)], axis=0)
    cb4 = conv_b.reshape(2, 5, 1, HD)
    ab2 = adapt_b.reshape(1, HD)
    tids2 = topic_ids.astype(jnp.int32).reshape(NTN, 1)
    y2 = y_data.reshape(NB, 1)
    ow2 = out_W.reshape(1, HD)
    ob2 = out_b.reshape(1, 1)

    m0, b0, m1, b1, t1, tb1, ht0 = _prep(adapt_W, ab2, conv_W, cb4, tids2, temb_pad)

    tword = _matmul_bias(word_embeds, m0, b0, tr=1000)
    hw0 = _gather_embeds(tword, wid_pad)

    at, atc = _agg_topic(wt[0], wt[1], wt[2], hw0, tt[0], tt[1], tt[2], ht0)
    aww, awwc = _agg_ww(ww[0], ww[1], ww[2], hw0)

    hw1 = _scale_relu_matmul(aww, awwc.reshape(CW, 1), m1, b1)
    ht1 = _combine_pair_matmul(at, atc.reshape(4, CT, 1), t1, tb1)

    ad, adc = _agg_doc(wd[0], wd[1], wd[2], hw1, td[0], td[1], td[2], ht1)

    loss2, yp = _final(ad, adc.reshape(4, CD, 1), ow2, ob2, y2)
    return (loss2.reshape(()), yp)
